# Initial kernel scaffold; baseline (speedup 1.0000x reference)
import jax
import jax.numpy as jnp
from jax.experimental import pallas as pl


def kernel(x, edge_index, Wl0, bl0, Wr0, g0, be0, Wl1, bl1, Wr1, g1, be1, Wh1, bh1, Wh2, bh2):
    raise NotImplementedError("write your pallas kernel here")


# speedup vs baseline: 6.0956x; 6.0956x over previous
import functools
import math

import jax
import jax.numpy as jnp
from jax import lax
from jax.experimental import pallas as pl
from jax.experimental.pallas import tpu as pltpu
from jax.experimental.pallas import tpu_sc as plsc

N, E, D, H = 10000, 320000, 128, 64
NPAD = 10240
NC, NS = 2, 16
NW = NC * NS
EW = E // NW
CS = 80
NCHUNK = EW // CS
RPT = NPAD // NS
INV_BN = 1.0 / math.sqrt(1.0 + 1e-5)

_mesh = plsc.VectorSubcoreMesh(core_axis_name="c", subcore_axis_name="s")


def _sc_agg_cnt_body(table, src, dst, agg_out, cnt_out,
                     srcv, dstv, rows, cntv, agg_sh, sem):
    c = lax.axis_index("c")
    s = lax.axis_index("s")
    wid = c * NS + s
    z16 = jnp.zeros((16,), jnp.float32)
    ones16 = jnp.ones((16,), jnp.float32)

    for r in range(CS):
        for j in range(H // 16):
            rows[r, pl.ds(j * 16, 16)] = z16

    def zdma(k, carry):
        pltpu.sync_copy(rows, agg_sh.at[pl.ds(s * RPT + k * CS, CS)])
        return carry
    lax.fori_loop(0, RPT // CS, zdma, 0)

    def zcnt(i, carry):
        cntv[pl.ds(i * 16, 16)] = z16
        return carry
    lax.fori_loop(0, NPAD // 16, zcnt, 0)

    plsc.subcore_barrier()

    def body(i, carry):
        base = wid * EW + i * CS
        pltpu.sync_copy(src.at[pl.ds(base, CS)], srcv)
        pltpu.sync_copy(dst.at[pl.ds(base, CS)], dstv)
        pltpu.async_copy(table.at[srcv], rows, sem).wait()
        pltpu.sync_copy(rows, agg_sh.at[dstv], add=True)
        for j in range(CS // 16):
            idxj = dstv[pl.ds(j * 16, 16)]
            plsc.addupdate_scatter(cntv, [idxj], ones16)
        return carry
    lax.fori_loop(0, NCHUNK, body, 0)

    plsc.subcore_barrier()
    pltpu.sync_copy(agg_sh.at[pl.ds(s * RPT, RPT)],
                    agg_out.at[pl.ds(c * NPAD + s * RPT, RPT)])
    pltpu.sync_copy(cntv, cnt_out.at[wid])


def _sc_agg_body(table, src, dst, agg_out, srcv, dstv, rows, agg_sh, sem):
    c = lax.axis_index("c")
    s = lax.axis_index("s")
    wid = c * NS + s
    z16 = jnp.zeros((16,), jnp.float32)

    for r in range(CS):
        for j in range(H // 16):
            rows[r, pl.ds(j * 16, 16)] = z16

    def zdma(k, carry):
        pltpu.sync_copy(rows, agg_sh.at[pl.ds(s * RPT + k * CS, CS)])
        return carry
    lax.fori_loop(0, RPT // CS, zdma, 0)

    plsc.subcore_barrier()

    def body(i, carry):
        base = wid * EW + i * CS
        pltpu.sync_copy(src.at[pl.ds(base, CS)], srcv)
        pltpu.sync_copy(dst.at[pl.ds(base, CS)], dstv)
        pltpu.async_copy(table.at[srcv], rows, sem).wait()
        pltpu.sync_copy(rows, agg_sh.at[dstv], add=True)
        return carry
    lax.fori_loop(0, NCHUNK, body, 0)

    plsc.subcore_barrier()
    pltpu.sync_copy(agg_sh.at[pl.ds(s * RPT, RPT)],
                    agg_out.at[pl.ds(c * NPAD + s * RPT, RPT)])


_sc_params = pltpu.CompilerParams(needs_layout_passes=False,
                                  use_tc_tiling_on_sc=False)

_sc_agg_cnt = pl.kernel(
    _sc_agg_cnt_body,
    compiler_params=_sc_params,
    out_type=(jax.ShapeDtypeStruct((NC * NPAD, H), jnp.float32),
              jax.ShapeDtypeStruct((NW, NPAD), jnp.float32)),
    mesh=_mesh,
    scratch_types=[
        pltpu.VMEM((CS,), jnp.int32),
        pltpu.VMEM((CS,), jnp.int32),
        pltpu.VMEM((CS, H), jnp.float32),
        pltpu.VMEM((NPAD,), jnp.float32),
        pltpu.VMEM_SHARED((NPAD, H), jnp.float32),
        pltpu.SemaphoreType.DMA,
    ],
)

_sc_agg = pl.kernel(
    _sc_agg_body,
    compiler_params=_sc_params,
    out_type=jax.ShapeDtypeStruct((NC * NPAD, H), jnp.float32),
    mesh=_mesh,
    scratch_types=[
        pltpu.VMEM((CS,), jnp.int32),
        pltpu.VMEM((CS,), jnp.int32),
        pltpu.VMEM((CS, H), jnp.float32),
        pltpu.VMEM_SHARED((NPAD, H), jnp.float32),
        pltpu.SemaphoreType.DMA,
    ],
)

BM = 2048
GRID = NPAD // BM


def _inproj_body(x_ref, wl_ref, wr_ref, y_ref, xr_ref):
    xb = x_ref[...]
    y_ref[...] = jnp.dot(xb, wl_ref[...], preferred_element_type=jnp.float32)
    xr_ref[...] = jnp.dot(xb, wr_ref[...], preferred_element_type=jnp.float32)


_tc_inproj = pl.pallas_call(
    _inproj_body,
    grid=(GRID,),
    in_specs=[pl.BlockSpec((BM, D), lambda i: (i, 0)),
              pl.BlockSpec((D, H), lambda i: (0, 0)),
              pl.BlockSpec((D, H), lambda i: (0, 0))],
    out_specs=[pl.BlockSpec((BM, H), lambda i: (i, 0)),
               pl.BlockSpec((BM, H), lambda i: (i, 0))],
    out_shape=[jax.ShapeDtypeStruct((NPAD, H), jnp.float32),
               jax.ShapeDtypeStruct((NPAD, H), jnp.float32)],
)


def _mid_body(agg_ref, cnt_ref, xr_ref, bl0_ref, g0_ref, be0_ref,
              wl1_ref, wr1_ref, y1_ref, hr1_ref):
    aggs = agg_ref[0] + agg_ref[1]
    cnt = jnp.sum(cnt_ref[...], axis=1, keepdims=True)
    inv = 1.0 / jnp.maximum(cnt, 1.0)
    pre = aggs * inv + bl0_ref[...] + xr_ref[...]
    h0 = jnp.maximum(pre * (g0_ref[...] * INV_BN) + be0_ref[...], 0.0)
    y1_ref[...] = jnp.dot(h0, wl1_ref[...], preferred_element_type=jnp.float32)
    hr1_ref[...] = jnp.dot(h0, wr1_ref[...], preferred_element_type=jnp.float32)


_tc_mid = pl.pallas_call(
    _mid_body,
    grid=(GRID,),
    in_specs=[pl.BlockSpec((NC, BM, H), lambda i: (0, i, 0)),
              pl.BlockSpec((BM, NW), lambda i: (i, 0)),
              pl.BlockSpec((BM, H), lambda i: (i, 0)),
              pl.BlockSpec((1, H), lambda i: (0, 0)),
              pl.BlockSpec((1, H), lambda i: (0, 0)),
              pl.BlockSpec((1, H), lambda i: (0, 0)),
              pl.BlockSpec((H, H), lambda i: (0, 0)),
              pl.BlockSpec((H, H), lambda i: (0, 0))],
    out_specs=[pl.BlockSpec((BM, H), lambda i: (i, 0)),
               pl.BlockSpec((BM, H), lambda i: (i, 0))],
    out_shape=[jax.ShapeDtypeStruct((NPAD, H), jnp.float32),
               jax.ShapeDtypeStruct((NPAD, H), jnp.float32)],
)


def _head_body(agg_ref, cnt_ref, hr_ref, bl1_ref, g1_ref, be1_ref,
               wh1_ref, bh1_ref, wh2_ref, bh2_ref, o_ref):
    aggs = agg_ref[0] + agg_ref[1]
    cnt = jnp.sum(cnt_ref[...], axis=1, keepdims=True)
    inv = 1.0 / jnp.maximum(cnt, 1.0)
    pre = aggs * inv + bl1_ref[...] + hr_ref[...]
    h1 = jnp.maximum(pre * (g1_ref[...] * INV_BN) + be1_ref[...], 0.0)
    z = jnp.maximum(
        jnp.dot(h1, wh1_ref[...], preferred_element_type=jnp.float32)
        + bh1_ref[...], 0.0)
    o = jnp.dot(z, wh2_ref[...], preferred_element_type=jnp.float32) + bh2_ref[...]
    o_ref[...] = jax.nn.sigmoid(o)


_tc_head = pl.pallas_call(
    _head_body,
    grid=(GRID,),
    in_specs=[pl.BlockSpec((NC, BM, H), lambda i: (0, i, 0)),
              pl.BlockSpec((BM, NW), lambda i: (i, 0)),
              pl.BlockSpec((BM, H), lambda i: (i, 0)),
              pl.BlockSpec((1, H), lambda i: (0, 0)),
              pl.BlockSpec((1, H), lambda i: (0, 0)),
              pl.BlockSpec((1, H), lambda i: (0, 0)),
              pl.BlockSpec((H, H // 2), lambda i: (0, 0)),
              pl.BlockSpec((1, H // 2), lambda i: (0, 0)),
              pl.BlockSpec((H // 2, 1), lambda i: (0, 0)),
              pl.BlockSpec((1, 1), lambda i: (0, 0))],
    out_specs=pl.BlockSpec((BM, 1), lambda i: (i, 0)),
    out_shape=jax.ShapeDtypeStruct((NPAD, 1), jnp.float32),
)


@jax.jit
def kernel(x, edge_index, Wl0, bl0, Wr0, g0, be0, Wl1, bl1, Wr1, g1, be1,
           Wh1, bh1, Wh2, bh2):
    src = edge_index[0].astype(jnp.int32)
    dst = edge_index[1].astype(jnp.int32)
    xp = jnp.pad(x, ((0, NPAD - N), (0, 0)))

    y0, xr0 = _tc_inproj(xp, Wl0, Wr0)
    agg0, cnt = _sc_agg_cnt(y0, src, dst)
    cnt_t = cnt.T
    y1, hr1 = _tc_mid(agg0.reshape(NC, NPAD, H), cnt_t, xr0,
                      bl0.reshape(1, H), g0.reshape(1, H), be0.reshape(1, H),
                      Wl1, Wr1)
    agg1 = _sc_agg(y1, src, dst)
    out = _tc_head(agg1.reshape(NC, NPAD, H), cnt_t, hr1,
                   bl1.reshape(1, H), g1.reshape(1, H), be1.reshape(1, H),
                   Wh1, bh1.reshape(1, H // 2), Wh2, bh2.reshape(1, 1))
    return out[:N, 0]


# speedup vs baseline: 13.4916x; 2.2133x over previous
import functools
import math

import jax
import jax.numpy as jnp
from jax import lax
from jax.experimental import pallas as pl
from jax.experimental.pallas import tpu as pltpu
from jax.experimental.pallas import tpu_sc as plsc

N, E, D, H = 10000, 320000, 128, 64
NPAD = 10240
NC, NS = 2, 16
NW = NC * NS
EW = E // NW
CS = 80
NCHUNK = EW // CS
RPT = NPAD // NS
INV_BN = 1.0 / math.sqrt(1.0 + 1e-5)

_mesh = plsc.VectorSubcoreMesh(core_axis_name="c", subcore_axis_name="s")


KB = 5
NSTEP = NCHUNK // KB


def _sc_agg_impl(with_cnt, table, src, dst, agg_out, cnt_out,
                 src_slab, dst_slab, rows, cntv, agg_sh, sems):
    c = lax.axis_index("c")
    s = lax.axis_index("s")
    wid = c * NS + s
    z16 = jnp.zeros((16,), jnp.float32)
    ones16 = jnp.ones((16,), jnp.float32)

    pltpu.sync_copy(src.at[pl.ds(wid * NCHUNK, NCHUNK)], src_slab)
    pltpu.sync_copy(dst.at[pl.ds(wid * NCHUNK, NCHUNK)], dst_slab)

    for r in range(CS):
        for j in range(H // 16):
            rows[0, r, pl.ds(j * 16, 16)] = z16

    def zdma(k, carry):
        pltpu.sync_copy(rows.at[0], agg_sh.at[pl.ds(s * RPT + k * CS, CS)])
        return carry
    lax.fori_loop(0, RPT // CS, zdma, 0)

    if with_cnt:
        def zcnt(i, carry):
            cntv[pl.ds(i * 16, 16)] = z16
            return carry
        lax.fori_loop(0, NPAD // 16, zcnt, 0)

    plsc.subcore_barrier()

    def body(i, carry):
        c0 = i * KB
        descs = [
            pltpu.async_copy(table.at[src_slab.at[c0 + j]], rows.at[j],
                             sems[j])
            for j in range(KB)
        ]
        for j in range(KB):
            descs[j].wait()
            pltpu.sync_copy(rows.at[j], agg_sh.at[dst_slab.at[c0 + j]],
                            add=True)
            if with_cnt:
                for k in range(CS // 16):
                    idxk = dst_slab[c0 + j, pl.ds(k * 16, 16)]
                    plsc.addupdate_scatter(cntv, [idxk], ones16)
        return carry
    lax.fori_loop(0, NSTEP, body, 0)

    plsc.subcore_barrier()
    pltpu.sync_copy(agg_sh.at[pl.ds(s * RPT, RPT)],
                    agg_out.at[pl.ds(c * NPAD + s * RPT, RPT)])
    if with_cnt:
        pltpu.sync_copy(cntv, cnt_out.at[wid])


def _sc_agg_cnt_body(table, src, dst, agg_out, cnt_out,
                     src_slab, dst_slab, rows, cntv, agg_sh, *sems):
    _sc_agg_impl(True, table, src, dst, agg_out, cnt_out,
                 src_slab, dst_slab, rows, cntv, agg_sh, sems)


def _sc_agg_body(table, src, dst, agg_out,
                 src_slab, dst_slab, rows, agg_sh, *sems):
    _sc_agg_impl(False, table, src, dst, agg_out, None,
                 src_slab, dst_slab, rows, None, agg_sh, sems)


_sc_params = pltpu.CompilerParams(needs_layout_passes=False,
                                  use_tc_tiling_on_sc=False)

_sc_agg_cnt = pl.kernel(
    _sc_agg_cnt_body,
    compiler_params=_sc_params,
    out_type=(jax.ShapeDtypeStruct((NC * NPAD, H), jnp.float32),
              jax.ShapeDtypeStruct((NW, NPAD), jnp.float32)),
    mesh=_mesh,
    scratch_types=[
        pltpu.VMEM((NCHUNK, CS), jnp.int32),
        pltpu.VMEM((NCHUNK, CS), jnp.int32),
        pltpu.VMEM((KB, CS, H), jnp.float32),
        pltpu.VMEM((NPAD,), jnp.float32),
        pltpu.VMEM_SHARED((NPAD, H), jnp.float32),
    ] + [pltpu.SemaphoreType.DMA] * KB,
)

_sc_agg = pl.kernel(
    _sc_agg_body,
    compiler_params=_sc_params,
    out_type=jax.ShapeDtypeStruct((NC * NPAD, H), jnp.float32),
    mesh=_mesh,
    scratch_types=[
        pltpu.VMEM((NCHUNK, CS), jnp.int32),
        pltpu.VMEM((NCHUNK, CS), jnp.int32),
        pltpu.VMEM((KB, CS, H), jnp.float32),
        pltpu.VMEM_SHARED((NPAD, H), jnp.float32),
    ] + [pltpu.SemaphoreType.DMA] * KB,
)

BM = 2048
GRID = NPAD // BM


def _inproj_body(x_ref, wl_ref, wr_ref, y_ref, xr_ref):
    xb = x_ref[...]
    y_ref[...] = jnp.dot(xb, wl_ref[...], preferred_element_type=jnp.float32)
    xr_ref[...] = jnp.dot(xb, wr_ref[...], preferred_element_type=jnp.float32)


_tc_inproj = pl.pallas_call(
    _inproj_body,
    grid=(GRID,),
    in_specs=[pl.BlockSpec((BM, D), lambda i: (i, 0)),
              pl.BlockSpec((D, H), lambda i: (0, 0)),
              pl.BlockSpec((D, H), lambda i: (0, 0))],
    out_specs=[pl.BlockSpec((BM, H), lambda i: (i, 0)),
               pl.BlockSpec((BM, H), lambda i: (i, 0))],
    out_shape=[jax.ShapeDtypeStruct((NPAD, H), jnp.float32),
               jax.ShapeDtypeStruct((NPAD, H), jnp.float32)],
)


def _mid_body(agg_ref, cnt_ref, xr_ref, bl0_ref, g0_ref, be0_ref,
              wl1_ref, wr1_ref, y1_ref, hr1_ref):
    aggs = agg_ref[0] + agg_ref[1]
    cnt = jnp.sum(cnt_ref[...], axis=1, keepdims=True)
    inv = 1.0 / jnp.maximum(cnt, 1.0)
    pre = aggs * inv + bl0_ref[...] + xr_ref[...]
    h0 = jnp.maximum(pre * (g0_ref[...] * INV_BN) + be0_ref[...], 0.0)
    y1_ref[...] = jnp.dot(h0, wl1_ref[...], preferred_element_type=jnp.float32)
    hr1_ref[...] = jnp.dot(h0, wr1_ref[...], preferred_element_type=jnp.float32)


_tc_mid = pl.pallas_call(
    _mid_body,
    grid=(GRID,),
    in_specs=[pl.BlockSpec((NC, BM, H), lambda i: (0, i, 0)),
              pl.BlockSpec((BM, NW), lambda i: (i, 0)),
              pl.BlockSpec((BM, H), lambda i: (i, 0)),
              pl.BlockSpec((1, H), lambda i: (0, 0)),
              pl.BlockSpec((1, H), lambda i: (0, 0)),
              pl.BlockSpec((1, H), lambda i: (0, 0)),
              pl.BlockSpec((H, H), lambda i: (0, 0)),
              pl.BlockSpec((H, H), lambda i: (0, 0))],
    out_specs=[pl.BlockSpec((BM, H), lambda i: (i, 0)),
               pl.BlockSpec((BM, H), lambda i: (i, 0))],
    out_shape=[jax.ShapeDtypeStruct((NPAD, H), jnp.float32),
               jax.ShapeDtypeStruct((NPAD, H), jnp.float32)],
)


def _head_body(agg_ref, cnt_ref, hr_ref, bl1_ref, g1_ref, be1_ref,
               wh1_ref, bh1_ref, wh2_ref, bh2_ref, o_ref):
    aggs = agg_ref[0] + agg_ref[1]
    cnt = jnp.sum(cnt_ref[...], axis=1, keepdims=True)
    inv = 1.0 / jnp.maximum(cnt, 1.0)
    pre = aggs * inv + bl1_ref[...] + hr_ref[...]
    h1 = jnp.maximum(pre * (g1_ref[...] * INV_BN) + be1_ref[...], 0.0)
    z = jnp.maximum(
        jnp.dot(h1, wh1_ref[...], preferred_element_type=jnp.float32)
        + bh1_ref[...], 0.0)
    o = jnp.dot(z, wh2_ref[...], preferred_element_type=jnp.float32) + bh2_ref[...]
    o_ref[...] = jax.nn.sigmoid(o)


_tc_head = pl.pallas_call(
    _head_body,
    grid=(GRID,),
    in_specs=[pl.BlockSpec((NC, BM, H), lambda i: (0, i, 0)),
              pl.BlockSpec((BM, NW), lambda i: (i, 0)),
              pl.BlockSpec((BM, H), lambda i: (i, 0)),
              pl.BlockSpec((1, H), lambda i: (0, 0)),
              pl.BlockSpec((1, H), lambda i: (0, 0)),
              pl.BlockSpec((1, H), lambda i: (0, 0)),
              pl.BlockSpec((H, H // 2), lambda i: (0, 0)),
              pl.BlockSpec((1, H // 2), lambda i: (0, 0)),
              pl.BlockSpec((H // 2, 1), lambda i: (0, 0)),
              pl.BlockSpec((1, 1), lambda i: (0, 0))],
    out_specs=pl.BlockSpec((BM, 1), lambda i: (i, 0)),
    out_shape=jax.ShapeDtypeStruct((NPAD, 1), jnp.float32),
)


@jax.jit
def kernel(x, edge_index, Wl0, bl0, Wr0, g0, be0, Wl1, bl1, Wr1, g1, be1,
           Wh1, bh1, Wh2, bh2):
    src = edge_index[0].astype(jnp.int32).reshape(NW * NCHUNK, CS)
    dst = edge_index[1].astype(jnp.int32).reshape(NW * NCHUNK, CS)
    xp = jnp.pad(x, ((0, NPAD - N), (0, 0)))

    y0, xr0 = _tc_inproj(xp, Wl0, Wr0)
    agg0, cnt = _sc_agg_cnt(y0, src, dst)
    cnt_t = cnt.T
    y1, hr1 = _tc_mid(agg0.reshape(NC, NPAD, H), cnt_t, xr0,
                      bl0.reshape(1, H), g0.reshape(1, H), be0.reshape(1, H),
                      Wl1, Wr1)
    agg1 = _sc_agg(y1, src, dst)
    out = _tc_head(agg1.reshape(NC, NPAD, H), cnt_t, hr1,
                   bl1.reshape(1, H), g1.reshape(1, H), be1.reshape(1, H),
                   Wh1, bh1.reshape(1, H // 2), Wh2, bh2.reshape(1, 1))
    return out[:N, 0]


# speedup vs baseline: 13.7042x; 1.0158x over previous
import functools
import math

import jax
import jax.numpy as jnp
from jax import lax
from jax.experimental import pallas as pl
from jax.experimental.pallas import tpu as pltpu
from jax.experimental.pallas import tpu_sc as plsc

N, E, D, H = 10000, 320000, 128, 64
NPAD = 10240
NC, NS = 2, 16
NW = NC * NS
EW = E // NW
CS = 80
NCHUNK = EW // CS
RPT = NPAD // NS
INV_BN = 1.0 / math.sqrt(1.0 + 1e-5)

_mesh = plsc.VectorSubcoreMesh(core_axis_name="c", subcore_axis_name="s")


KB = 5
NSTEP = NCHUNK // KB


def _sc_agg_impl(with_cnt, table, src, dst, agg_out, cnt_out,
                 src_slab, dst_slab, rows, cntv, agg_sh, sems):
    c = lax.axis_index("c")
    s = lax.axis_index("s")
    wid = c * NS + s
    z16 = jnp.zeros((16,), jnp.float32)
    ones16 = jnp.ones((16,), jnp.float32)

    pltpu.sync_copy(src.at[pl.ds(wid * NCHUNK, NCHUNK)], src_slab)
    pltpu.sync_copy(dst.at[pl.ds(wid * NCHUNK, NCHUNK)], dst_slab)

    for r in range(CS):
        for j in range(H // 16):
            rows[0, r, pl.ds(j * 16, 16)] = z16

    def zdma(k, carry):
        pltpu.sync_copy(rows.at[0], agg_sh.at[pl.ds(s * RPT + k * CS, CS)])
        return carry
    lax.fori_loop(0, RPT // CS, zdma, 0)

    if with_cnt:
        def zcnt(i, carry):
            cntv[pl.ds(i * 16, 16)] = z16
            return carry
        lax.fori_loop(0, NPAD // 16, zcnt, 0)

    plsc.subcore_barrier()

    def body(i, carry):
        c0 = i * KB
        descs = [
            pltpu.async_copy(table.at[src_slab.at[c0 + j]], rows.at[j],
                             sems[j])
            for j in range(KB)
        ]
        for j in range(KB):
            descs[j].wait()
            pltpu.sync_copy(rows.at[j], agg_sh.at[dst_slab.at[c0 + j]],
                            add=True)
            if with_cnt:
                for k in range(CS // 16):
                    idxk = dst_slab[c0 + j, pl.ds(k * 16, 16)]
                    plsc.addupdate_scatter(cntv, [idxk], ones16)
        return carry
    lax.fori_loop(0, NSTEP, body, 0)

    plsc.subcore_barrier()
    pltpu.sync_copy(agg_sh.at[pl.ds(s * RPT, RPT)],
                    agg_out.at[pl.ds(c * NPAD + s * RPT, RPT)])
    if with_cnt:
        pltpu.sync_copy(cntv, cnt_out.at[wid])


def _sc_agg_cnt_body(table, src, dst, agg_out, cnt_out,
                     src_slab, dst_slab, rows, cntv, agg_sh, *sems):
    _sc_agg_impl(True, table, src, dst, agg_out, cnt_out,
                 src_slab, dst_slab, rows, cntv, agg_sh, sems)


def _sc_agg_body(table, src, dst, agg_out,
                 src_slab, dst_slab, rows, agg_sh, *sems):
    _sc_agg_impl(False, table, src, dst, agg_out, None,
                 src_slab, dst_slab, rows, None, agg_sh, sems)


_sc_params = pltpu.CompilerParams(needs_layout_passes=False,
                                  use_tc_tiling_on_sc=False)

_sc_agg_cnt = pl.kernel(
    _sc_agg_cnt_body,
    compiler_params=_sc_params,
    out_type=(jax.ShapeDtypeStruct((NC * NPAD, H), jnp.float32),
              jax.ShapeDtypeStruct((NW, NPAD), jnp.float32)),
    mesh=_mesh,
    scratch_types=[
        pltpu.VMEM((NCHUNK, CS), jnp.int32),
        pltpu.VMEM((NCHUNK, CS), jnp.int32),
        pltpu.VMEM((KB, CS, H), jnp.float32),
        pltpu.VMEM((NPAD,), jnp.float32),
        pltpu.VMEM_SHARED((NPAD, H), jnp.float32),
    ] + [pltpu.SemaphoreType.DMA] * KB,
)

_sc_agg = pl.kernel(
    _sc_agg_body,
    compiler_params=_sc_params,
    out_type=jax.ShapeDtypeStruct((NC * NPAD, H), jnp.float32),
    mesh=_mesh,
    scratch_types=[
        pltpu.VMEM((NCHUNK, CS), jnp.int32),
        pltpu.VMEM((NCHUNK, CS), jnp.int32),
        pltpu.VMEM((KB, CS, H), jnp.float32),
        pltpu.VMEM_SHARED((NPAD, H), jnp.float32),
    ] + [pltpu.SemaphoreType.DMA] * KB,
)

BM = 2000
GRID = N // BM


def _proj_body(x_ref, w_ref, y_ref):
    y_ref[...] = jnp.dot(x_ref[...], w_ref[...],
                         preferred_element_type=jnp.float32)


def _make_proj(din):
    return pl.pallas_call(
        _proj_body,
        grid=(GRID,),
        in_specs=[pl.BlockSpec((BM, din), lambda i: (i, 0)),
                  pl.BlockSpec((din, H), lambda i: (0, 0))],
        out_specs=pl.BlockSpec((BM, H), lambda i: (i, 0)),
        out_shape=jax.ShapeDtypeStruct((N, H), jnp.float32),
    )


_tc_proj_d = _make_proj(D)
_tc_proj_h = _make_proj(H)


def _mid_body(agg_ref, cnt_ref, xr_ref, bl0_ref, g0_ref, be0_ref,
              wl1_ref, y1_ref, h0_ref):
    aggs = agg_ref[0] + agg_ref[1]
    cnt = jnp.sum(cnt_ref[...], axis=1, keepdims=True)
    inv = 1.0 / jnp.maximum(cnt, 1.0)
    pre = aggs * inv + bl0_ref[...] + xr_ref[...]
    h0 = jnp.maximum(pre * (g0_ref[...] * INV_BN) + be0_ref[...], 0.0)
    h0_ref[...] = h0
    y1_ref[...] = jnp.dot(h0, wl1_ref[...], preferred_element_type=jnp.float32)


_tc_mid = pl.pallas_call(
    _mid_body,
    grid=(GRID,),
    in_specs=[pl.BlockSpec((NC, BM, H), lambda i: (0, i, 0)),
              pl.BlockSpec((BM, NW), lambda i: (i, 0)),
              pl.BlockSpec((BM, H), lambda i: (i, 0)),
              pl.BlockSpec((1, H), lambda i: (0, 0)),
              pl.BlockSpec((1, H), lambda i: (0, 0)),
              pl.BlockSpec((1, H), lambda i: (0, 0)),
              pl.BlockSpec((H, H), lambda i: (0, 0))],
    out_specs=[pl.BlockSpec((BM, H), lambda i: (i, 0)),
               pl.BlockSpec((BM, H), lambda i: (i, 0))],
    out_shape=[jax.ShapeDtypeStruct((N, H), jnp.float32),
               jax.ShapeDtypeStruct((N, H), jnp.float32)],
)


def _head_body(agg_ref, cnt_ref, hr_ref, bl1_ref, g1_ref, be1_ref,
               wh1_ref, bh1_ref, wh2_ref, bh2_ref, o_ref):
    aggs = agg_ref[0] + agg_ref[1]
    cnt = jnp.sum(cnt_ref[...], axis=1, keepdims=True)
    inv = 1.0 / jnp.maximum(cnt, 1.0)
    pre = aggs * inv + bl1_ref[...] + hr_ref[...]
    h1 = jnp.maximum(pre * (g1_ref[...] * INV_BN) + be1_ref[...], 0.0)
    z = jnp.maximum(
        jnp.dot(h1, wh1_ref[...], preferred_element_type=jnp.float32)
        + bh1_ref[...], 0.0)
    o = jnp.dot(z, wh2_ref[...], preferred_element_type=jnp.float32) + bh2_ref[...]
    o_ref[...] = jax.nn.sigmoid(o)


_tc_head = pl.pallas_call(
    _head_body,
    grid=(GRID,),
    in_specs=[pl.BlockSpec((NC, BM, H), lambda i: (0, i, 0)),
              pl.BlockSpec((BM, NW), lambda i: (i, 0)),
              pl.BlockSpec((BM, H), lambda i: (i, 0)),
              pl.BlockSpec((1, H), lambda i: (0, 0)),
              pl.BlockSpec((1, H), lambda i: (0, 0)),
              pl.BlockSpec((1, H), lambda i: (0, 0)),
              pl.BlockSpec((H, H // 2), lambda i: (0, 0)),
              pl.BlockSpec((1, H // 2), lambda i: (0, 0)),
              pl.BlockSpec((H // 2, 1), lambda i: (0, 0)),
              pl.BlockSpec((1, 1), lambda i: (0, 0))],
    out_specs=pl.BlockSpec((BM, 1), lambda i: (i, 0)),
    out_shape=jax.ShapeDtypeStruct((N, 1), jnp.float32),
)


@jax.jit
def kernel(x, edge_index, Wl0, bl0, Wr0, g0, be0, Wl1, bl1, Wr1, g1, be1,
           Wh1, bh1, Wh2, bh2):
    src = edge_index[0].astype(jnp.int32).reshape(NW * NCHUNK, CS)
    dst = edge_index[1].astype(jnp.int32).reshape(NW * NCHUNK, CS)

    y0 = _tc_proj_d(x, Wl0)
    agg0, cnt = _sc_agg_cnt(y0, src, dst)
    xr0 = _tc_proj_d(x, Wr0)
    cnt_t = cnt.T
    y1, h0 = _tc_mid(agg0.reshape(NC, NPAD, H), cnt_t, xr0,
                     bl0.reshape(1, H), g0.reshape(1, H), be0.reshape(1, H),
                     Wl1)
    agg1 = _sc_agg(y1, src, dst)
    hr1 = _tc_proj_h(h0, Wr1)
    out = _tc_head(agg1.reshape(NC, NPAD, H), cnt_t, hr1,
                   bl1.reshape(1, H), g1.reshape(1, H), be1.reshape(1, H),
                   Wh1, bh1.reshape(1, H // 2), Wh2, bh2.reshape(1, 1))
    return out[:, 0]


# speedup vs baseline: 15.5886x; 1.1375x over previous
import functools
import math

import jax
import jax.numpy as jnp
from jax import lax
from jax.experimental import pallas as pl
from jax.experimental.pallas import tpu as pltpu
from jax.experimental.pallas import tpu_sc as plsc

N, E, D, H = 10000, 320000, 128, 64
NPAD = 10240
NC, NS = 2, 16
NW = NC * NS
EW = E // NW
CS = 80
NCHUNK = EW // CS
RPT = NPAD // NS
INV_BN = 1.0 / math.sqrt(1.0 + 1e-5)

_mesh = plsc.VectorSubcoreMesh(core_axis_name="c", subcore_axis_name="s")


KB = 5
NSTEP = NCHUNK // KB


def _sc_agg_impl(with_cnt, table, src, dst, agg_out, cnt_out,
                 src_slab, dst_slab, rows, cntv, agg_sh, sems):
    c = lax.axis_index("c")
    s = lax.axis_index("s")
    wid = c * NS + s
    z16 = jnp.zeros((16,), jnp.float32)
    ones16 = jnp.ones((16,), jnp.float32)

    pltpu.sync_copy(src.at[pl.ds(wid * NCHUNK, NCHUNK)], src_slab)
    pltpu.sync_copy(dst.at[pl.ds(wid * NCHUNK, NCHUNK)], dst_slab)

    for r in range(CS):
        for j in range(H // 16):
            rows[0, r, pl.ds(j * 16, 16)] = z16

    def zdma(k, carry):
        pltpu.sync_copy(rows.at[0], agg_sh.at[pl.ds(s * RPT + k * CS, CS)])
        return carry
    lax.fori_loop(0, RPT // CS, zdma, 0)

    if with_cnt:
        def zcnt(i, carry):
            cntv[pl.ds(i * 16, 16)] = z16
            return carry
        lax.fori_loop(0, NPAD // 16, zcnt, 0)

    plsc.subcore_barrier()

    def body(i, carry):
        c0 = i * KB
        descs = [
            pltpu.async_copy(table.at[src_slab.at[c0 + j]], rows.at[j],
                             sems[j])
            for j in range(KB)
        ]
        for j in range(KB):
            descs[j].wait()
            pltpu.sync_copy(rows.at[j], agg_sh.at[dst_slab.at[c0 + j]],
                            add=True)
            if with_cnt:
                for k in range(CS // 16):
                    idxk = dst_slab[c0 + j, pl.ds(k * 16, 16)]
                    plsc.addupdate_scatter(cntv, [idxk], ones16)
        return carry
    lax.fori_loop(0, NSTEP, body, 0)

    plsc.subcore_barrier()
    pltpu.sync_copy(agg_sh.at[pl.ds(s * RPT, RPT)],
                    agg_out.at[pl.ds(c * NPAD + s * RPT, RPT)])
    if with_cnt:
        pltpu.sync_copy(cntv, cnt_out.at[wid])


def _sc_agg_cnt_body(table, src, dst, agg_out, cnt_out,
                     src_slab, dst_slab, rows, cntv, agg_sh, *sems):
    _sc_agg_impl(True, table, src, dst, agg_out, cnt_out,
                 src_slab, dst_slab, rows, cntv, agg_sh, sems)


def _sc_agg_body(table, src, dst, agg_out,
                 src_slab, dst_slab, rows, agg_sh, *sems):
    _sc_agg_impl(False, table, src, dst, agg_out, None,
                 src_slab, dst_slab, rows, None, agg_sh, sems)


_sc_params = pltpu.CompilerParams(needs_layout_passes=False,
                                  use_tc_tiling_on_sc=False)

_sc_agg_cnt = pl.kernel(
    _sc_agg_cnt_body,
    compiler_params=_sc_params,
    out_type=(jax.ShapeDtypeStruct((NC * NPAD, H), jnp.float32),
              jax.ShapeDtypeStruct((NW, NPAD), jnp.float32)),
    mesh=_mesh,
    scratch_types=[
        pltpu.VMEM((NCHUNK, CS), jnp.int32),
        pltpu.VMEM((NCHUNK, CS), jnp.int32),
        pltpu.VMEM((KB, CS, H), jnp.float32),
        pltpu.VMEM((NPAD,), jnp.float32),
        pltpu.VMEM_SHARED((NPAD, H), jnp.float32),
    ] + [pltpu.SemaphoreType.DMA] * KB,
)

_sc_agg = pl.kernel(
    _sc_agg_body,
    compiler_params=_sc_params,
    out_type=jax.ShapeDtypeStruct((NC * NPAD, H), jnp.float32),
    mesh=_mesh,
    scratch_types=[
        pltpu.VMEM((NCHUNK, CS), jnp.int32),
        pltpu.VMEM((NCHUNK, CS), jnp.int32),
        pltpu.VMEM((KB, CS, H), jnp.float32),
        pltpu.VMEM_SHARED((NPAD, H), jnp.float32),
    ] + [pltpu.SemaphoreType.DMA] * KB,
)

P = 2
PD, PH = P * D, P * H
NR = N // P
NPR = NPAD // P
BMP = 1000
GRID = NR // BMP


def _proj_body(x_ref, w_ref, y_ref):
    y_ref[...] = jnp.dot(x_ref[...], w_ref[...],
                         preferred_element_type=jnp.float32)


def _make_proj(din):
    return pl.pallas_call(
        _proj_body,
        grid=(GRID,),
        in_specs=[pl.BlockSpec((BMP, din), lambda i: (i, 0)),
                  pl.BlockSpec((din, PH), lambda i: (0, 0))],
        out_specs=pl.BlockSpec((BMP, PH), lambda i: (i, 0)),
        out_shape=jax.ShapeDtypeStruct((NR, PH), jnp.float32),
    )


_tc_proj_d = _make_proj(PD)
_tc_proj_h = _make_proj(PH)


def _mid_body(agg_ref, inv_ref, xr_ref, bl0_ref, g0_ref, be0_ref,
              wl1_ref, y1_ref, h0_ref):
    aggs = agg_ref[0] + agg_ref[1]
    pre = aggs * inv_ref[...] + bl0_ref[...] + xr_ref[...]
    h0 = jnp.maximum(pre * (g0_ref[...] * INV_BN) + be0_ref[...], 0.0)
    h0_ref[...] = h0
    y1_ref[...] = jnp.dot(h0, wl1_ref[...], preferred_element_type=jnp.float32)


_tc_mid = pl.pallas_call(
    _mid_body,
    grid=(GRID,),
    in_specs=[pl.BlockSpec((NC, BMP, PH), lambda i: (0, i, 0)),
              pl.BlockSpec((BMP, PH), lambda i: (i, 0)),
              pl.BlockSpec((BMP, PH), lambda i: (i, 0)),
              pl.BlockSpec((1, PH), lambda i: (0, 0)),
              pl.BlockSpec((1, PH), lambda i: (0, 0)),
              pl.BlockSpec((1, PH), lambda i: (0, 0)),
              pl.BlockSpec((PH, PH), lambda i: (0, 0))],
    out_specs=[pl.BlockSpec((BMP, PH), lambda i: (i, 0)),
               pl.BlockSpec((BMP, PH), lambda i: (i, 0))],
    out_shape=[jax.ShapeDtypeStruct((NR, PH), jnp.float32),
               jax.ShapeDtypeStruct((NR, PH), jnp.float32)],
)


def _head_body(agg_ref, inv_ref, hr_ref, bl1_ref, g1_ref, be1_ref,
               wh1_ref, bh1_ref, wh2_ref, bh2_ref, o_ref):
    aggs = agg_ref[0] + agg_ref[1]
    pre = aggs * inv_ref[...] + bl1_ref[...] + hr_ref[...]
    h1 = jnp.maximum(pre * (g1_ref[...] * INV_BN) + be1_ref[...], 0.0)
    z = jnp.maximum(
        jnp.dot(h1, wh1_ref[...], preferred_element_type=jnp.float32)
        + bh1_ref[...], 0.0)
    o = jnp.dot(z, wh2_ref[...], preferred_element_type=jnp.float32) + bh2_ref[...]
    o_ref[...] = jax.nn.sigmoid(o)


_tc_head = pl.pallas_call(
    _head_body,
    grid=(GRID,),
    in_specs=[pl.BlockSpec((NC, BMP, PH), lambda i: (0, i, 0)),
              pl.BlockSpec((BMP, PH), lambda i: (i, 0)),
              pl.BlockSpec((BMP, PH), lambda i: (i, 0)),
              pl.BlockSpec((1, PH), lambda i: (0, 0)),
              pl.BlockSpec((1, PH), lambda i: (0, 0)),
              pl.BlockSpec((1, PH), lambda i: (0, 0)),
              pl.BlockSpec((PH, P * (H // 2)), lambda i: (0, 0)),
              pl.BlockSpec((1, P * (H // 2)), lambda i: (0, 0)),
              pl.BlockSpec((P * (H // 2), P), lambda i: (0, 0)),
              pl.BlockSpec((1, P), lambda i: (0, 0))],
    out_specs=pl.BlockSpec((BMP, P), lambda i: (i, 0)),
    out_shape=jax.ShapeDtypeStruct((NR, P), jnp.float32),
)


def _blkdiag(w):
    din, dout = w.shape
    z = jnp.zeros((P, din, P, dout), w.dtype)
    z = z.at[0, :, 0, :].set(w).at[1, :, 1, :].set(w)
    return z.reshape(P * din, P * dout)


@jax.jit
def kernel(x, edge_index, Wl0, bl0, Wr0, g0, be0, Wl1, bl1, Wr1, g1, be1,
           Wh1, bh1, Wh2, bh2):
    src = edge_index[0].astype(jnp.int32).reshape(NW * NCHUNK, CS)
    dst = edge_index[1].astype(jnp.int32).reshape(NW * NCHUNK, CS)
    xp = x.reshape(NR, PD)

    y0 = _tc_proj_d(xp, _blkdiag(Wl0))
    agg0, cnt = _sc_agg_cnt(y0.reshape(N, H), src, dst)
    xr0 = _tc_proj_d(xp, _blkdiag(Wr0))
    inv = 1.0 / jnp.maximum(jnp.sum(cnt, axis=0), 1.0)
    inv_pf = jnp.broadcast_to(inv.reshape(NPR, P, 1),
                              (NPR, P, H)).reshape(NPR, PH)
    y1, h0 = _tc_mid(agg0.reshape(NC, NPR, PH), inv_pf, xr0,
                     jnp.tile(bl0, P).reshape(1, PH),
                     jnp.tile(g0, P).reshape(1, PH),
                     jnp.tile(be0, P).reshape(1, PH),
                     _blkdiag(Wl1))
    agg1 = _sc_agg(y1.reshape(N, H), src, dst)
    hr1 = _tc_proj_h(h0, _blkdiag(Wr1))
    out = _tc_head(agg1.reshape(NC, NPR, PH), inv_pf, hr1,
                   jnp.tile(bl1, P).reshape(1, PH),
                   jnp.tile(g1, P).reshape(1, PH),
                   jnp.tile(be1, P).reshape(1, PH),
                   _blkdiag(Wh1), jnp.tile(bh1, P).reshape(1, P * (H // 2)),
                   _blkdiag(Wh2), jnp.tile(bh2, P).reshape(1, P))
    return out.reshape(N)


# speedup vs baseline: 15.6224x; 1.0022x over previous
import functools
import math

import jax
import jax.numpy as jnp
from jax import lax
from jax.experimental import pallas as pl
from jax.experimental.pallas import tpu as pltpu
from jax.experimental.pallas import tpu_sc as plsc

N, E, D, H = 10000, 320000, 128, 64
NPAD = 10240
NC, NS = 2, 16
NW = NC * NS
EW = E // NW
CS = 80
NCHUNK = EW // CS
RPT = NPAD // NS
INV_BN = 1.0 / math.sqrt(1.0 + 1e-5)

_mesh = plsc.VectorSubcoreMesh(core_axis_name="c", subcore_axis_name="s")


KB = 5
NSTEP = NCHUNK // KB


def _sc_agg_impl(with_cnt, table, src, dst, agg_out, cnt_out,
                 src_slab, dst_slab, rows, cntv, agg_sh, sems):
    c = lax.axis_index("c")
    s = lax.axis_index("s")
    wid = c * NS + s
    z16 = jnp.zeros((16,), jnp.float32)
    ones16 = jnp.ones((16,), jnp.float32)

    pltpu.sync_copy(src.at[pl.ds(wid * NCHUNK, NCHUNK)], src_slab)
    pltpu.sync_copy(dst.at[pl.ds(wid * NCHUNK, NCHUNK)], dst_slab)

    for r in range(CS):
        for j in range(H // 16):
            rows[0, r, pl.ds(j * 16, 16)] = z16

    def zdma(k, carry):
        pltpu.sync_copy(rows.at[0], agg_sh.at[pl.ds(s * RPT + k * CS, CS)])
        return carry
    lax.fori_loop(0, RPT // CS, zdma, 0)

    if with_cnt:
        def zcnt(i, carry):
            cntv[pl.ds(i * 16, 16)] = z16
            return carry
        lax.fori_loop(0, NPAD // 16, zcnt, 0)

    plsc.subcore_barrier()

    def body(i, carry):
        c0 = i * KB
        descs = [
            pltpu.async_copy(table.at[src_slab.at[c0 + j]], rows.at[j],
                             sems[j])
            for j in range(KB)
        ]
        for j in range(KB):
            descs[j].wait()
            pltpu.sync_copy(rows.at[j], agg_sh.at[dst_slab.at[c0 + j]],
                            add=True)
            if with_cnt:
                for k in range(CS // 16):
                    idxk = dst_slab[c0 + j, pl.ds(k * 16, 16)]
                    plsc.addupdate_scatter(cntv, [idxk], ones16)
        return carry
    lax.fori_loop(0, NSTEP, body, 0)

    plsc.subcore_barrier()
    pltpu.sync_copy(agg_sh.at[pl.ds(s * RPT, RPT)],
                    agg_out.at[pl.ds(c * NPAD + s * RPT, RPT)])
    if with_cnt:
        pltpu.sync_copy(cntv, cnt_out.at[wid])


def _sc_agg_cnt_body(table, src, dst, agg_out, cnt_out,
                     src_slab, dst_slab, rows, cntv, agg_sh, *sems):
    _sc_agg_impl(True, table, src, dst, agg_out, cnt_out,
                 src_slab, dst_slab, rows, cntv, agg_sh, sems)


def _sc_agg_body(table, src, dst, agg_out,
                 src_slab, dst_slab, rows, agg_sh, *sems):
    _sc_agg_impl(False, table, src, dst, agg_out, None,
                 src_slab, dst_slab, rows, None, agg_sh, sems)


_sc_params = pltpu.CompilerParams(needs_layout_passes=False,
                                  use_tc_tiling_on_sc=False)

_sc_agg_cnt = pl.kernel(
    _sc_agg_cnt_body,
    compiler_params=_sc_params,
    out_type=(jax.ShapeDtypeStruct((NC * NPAD, H), jnp.float32),
              jax.ShapeDtypeStruct((NW, NPAD), jnp.float32)),
    mesh=_mesh,
    scratch_types=[
        pltpu.VMEM((NCHUNK, CS), jnp.int32),
        pltpu.VMEM((NCHUNK, CS), jnp.int32),
        pltpu.VMEM((KB, CS, H), jnp.float32),
        pltpu.VMEM((NPAD,), jnp.float32),
        pltpu.VMEM_SHARED((NPAD, H), jnp.float32),
    ] + [pltpu.SemaphoreType.DMA] * KB,
)

_sc_agg = pl.kernel(
    _sc_agg_body,
    compiler_params=_sc_params,
    out_type=jax.ShapeDtypeStruct((NC * NPAD, H), jnp.float32),
    mesh=_mesh,
    scratch_types=[
        pltpu.VMEM((NCHUNK, CS), jnp.int32),
        pltpu.VMEM((NCHUNK, CS), jnp.int32),
        pltpu.VMEM((KB, CS, H), jnp.float32),
        pltpu.VMEM_SHARED((NPAD, H), jnp.float32),
    ] + [pltpu.SemaphoreType.DMA] * KB,
)

P = 2
PD, PH = P * D, P * H
NR = N // P
NPR = NPAD // P
BMP = 1000
GRID = NR // BMP


def _proj_body(x_ref, w_ref, y_ref):
    y_ref[...] = jnp.dot(x_ref[...], w_ref[...],
                         preferred_element_type=jnp.float32)


def _make_proj(din):
    return pl.pallas_call(
        _proj_body,
        grid=(GRID,),
        in_specs=[pl.BlockSpec((BMP, din), lambda i: (i, 0)),
                  pl.BlockSpec((din, PH), lambda i: (0, 0))],
        out_specs=pl.BlockSpec((BMP, PH), lambda i: (i, 0)),
        out_shape=jax.ShapeDtypeStruct((NR, PH), jnp.float32),
    )


_tc_proj_d = _make_proj(PD)
_tc_proj_h = _make_proj(PH)


def _lane_select(inv2):
    lane = lax.broadcasted_iota(jnp.int32, (BMP, PH), 1)
    return jnp.where(lane < H, inv2[:, 0:1], inv2[:, 1:2])


def _mid_body(agg_ref, inv_ref, xr_ref, bl0_ref, g0_ref, be0_ref,
              wl1_ref, y1_ref, h0_ref):
    aggs = agg_ref[0] + agg_ref[1]
    pre = aggs * _lane_select(inv_ref[...]) + bl0_ref[...] + xr_ref[...]
    h0 = jnp.maximum(pre * (g0_ref[...] * INV_BN) + be0_ref[...], 0.0)
    h0_ref[...] = h0
    y1_ref[...] = jnp.dot(h0, wl1_ref[...], preferred_element_type=jnp.float32)


_tc_mid = pl.pallas_call(
    _mid_body,
    grid=(GRID,),
    in_specs=[pl.BlockSpec((NC, BMP, PH), lambda i: (0, i, 0)),
              pl.BlockSpec((BMP, P), lambda i: (i, 0)),
              pl.BlockSpec((BMP, PH), lambda i: (i, 0)),
              pl.BlockSpec((1, PH), lambda i: (0, 0)),
              pl.BlockSpec((1, PH), lambda i: (0, 0)),
              pl.BlockSpec((1, PH), lambda i: (0, 0)),
              pl.BlockSpec((PH, PH), lambda i: (0, 0))],
    out_specs=[pl.BlockSpec((BMP, PH), lambda i: (i, 0)),
               pl.BlockSpec((BMP, PH), lambda i: (i, 0))],
    out_shape=[jax.ShapeDtypeStruct((NR, PH), jnp.float32),
               jax.ShapeDtypeStruct((NR, PH), jnp.float32)],
)


def _head_body(agg_ref, inv_ref, hr_ref, bl1_ref, g1_ref, be1_ref,
               wh1_ref, bh1_ref, wh2_ref, bh2_ref, o_ref):
    aggs = agg_ref[0] + agg_ref[1]
    pre = aggs * _lane_select(inv_ref[...]) + bl1_ref[...] + hr_ref[...]
    h1 = jnp.maximum(pre * (g1_ref[...] * INV_BN) + be1_ref[...], 0.0)
    z = jnp.maximum(
        jnp.dot(h1, wh1_ref[...], preferred_element_type=jnp.float32)
        + bh1_ref[...], 0.0)
    o = jnp.dot(z, wh2_ref[...], preferred_element_type=jnp.float32) + bh2_ref[...]
    o_ref[...] = jax.nn.sigmoid(o)


_tc_head = pl.pallas_call(
    _head_body,
    grid=(GRID,),
    in_specs=[pl.BlockSpec((NC, BMP, PH), lambda i: (0, i, 0)),
              pl.BlockSpec((BMP, P), lambda i: (i, 0)),
              pl.BlockSpec((BMP, PH), lambda i: (i, 0)),
              pl.BlockSpec((1, PH), lambda i: (0, 0)),
              pl.BlockSpec((1, PH), lambda i: (0, 0)),
              pl.BlockSpec((1, PH), lambda i: (0, 0)),
              pl.BlockSpec((PH, P * (H // 2)), lambda i: (0, 0)),
              pl.BlockSpec((1, P * (H // 2)), lambda i: (0, 0)),
              pl.BlockSpec((P * (H // 2), P), lambda i: (0, 0)),
              pl.BlockSpec((1, P), lambda i: (0, 0))],
    out_specs=pl.BlockSpec((BMP, P), lambda i: (i, 0)),
    out_shape=jax.ShapeDtypeStruct((NR, P), jnp.float32),
)


def _blkdiag(w):
    din, dout = w.shape
    z = jnp.zeros((P, din, P, dout), w.dtype)
    z = z.at[0, :, 0, :].set(w).at[1, :, 1, :].set(w)
    return z.reshape(P * din, P * dout)


@jax.jit
def kernel(x, edge_index, Wl0, bl0, Wr0, g0, be0, Wl1, bl1, Wr1, g1, be1,
           Wh1, bh1, Wh2, bh2):
    src = edge_index[0].astype(jnp.int32).reshape(NW * NCHUNK, CS)
    dst = edge_index[1].astype(jnp.int32).reshape(NW * NCHUNK, CS)
    xp = x.reshape(NR, PD)

    y0 = _tc_proj_d(xp, _blkdiag(Wl0))
    agg0, cnt = _sc_agg_cnt(y0.reshape(N, H), src, dst)
    xr0 = _tc_proj_d(xp, _blkdiag(Wr0))
    inv2 = 1.0 / jnp.maximum(jnp.sum(cnt.reshape(NW, NPR, P), axis=0),
                             1.0)
    y1, h0 = _tc_mid(agg0.reshape(NC, NPR, PH), inv2, xr0,
                     jnp.tile(bl0, P).reshape(1, PH),
                     jnp.tile(g0, P).reshape(1, PH),
                     jnp.tile(be0, P).reshape(1, PH),
                     _blkdiag(Wl1))
    agg1 = _sc_agg(y1.reshape(N, H), src, dst)
    hr1 = _tc_proj_h(h0, _blkdiag(Wr1))
    out = _tc_head(agg1.reshape(NC, NPR, PH), inv2, hr1,
                   jnp.tile(bl1, P).reshape(1, PH),
                   jnp.tile(g1, P).reshape(1, PH),
                   jnp.tile(be1, P).reshape(1, PH),
                   _blkdiag(Wh1), jnp.tile(bh1, P).reshape(1, P * (H // 2)),
                   _blkdiag(Wh2), jnp.tile(bh2, P).reshape(1, P))
    return out.reshape(N)


# speedup vs baseline: 16.7085x; 1.0695x over previous
import functools
import math

import jax
import jax.numpy as jnp
from jax import lax
from jax.experimental import pallas as pl
from jax.experimental.pallas import tpu as pltpu
from jax.experimental.pallas import tpu_sc as plsc

N, E, D, H = 10000, 320000, 128, 64
NPAD = 10240
NC, NS = 2, 16
NW = NC * NS
EW = E // NW
CS = 80
NCHUNK = EW // CS
RPT = NPAD // NS
INV_BN = 1.0 / math.sqrt(1.0 + 1e-5)

_mesh = plsc.VectorSubcoreMesh(core_axis_name="c", subcore_axis_name="s")


KB = 5
NSTEP = NCHUNK // KB


def _sc_agg_impl(with_cnt, table, src, dst, agg_out, cnt_out,
                 src_slab, dst_slab, rows, cntv, agg_sh, sems):
    c = lax.axis_index("c")
    s = lax.axis_index("s")
    wid = c * NS + s
    z16 = jnp.zeros((16,), jnp.float32)
    ones16 = jnp.ones((16,), jnp.float32)

    pltpu.sync_copy(src.at[pl.ds(wid * NCHUNK, NCHUNK)], src_slab)
    pltpu.sync_copy(dst.at[pl.ds(wid * NCHUNK, NCHUNK)], dst_slab)

    for r in range(CS):
        for j in range(H // 16):
            rows[0, r, pl.ds(j * 16, 16)] = z16

    def zdma(k, carry):
        pltpu.sync_copy(rows.at[0], agg_sh.at[pl.ds(s * RPT + k * CS, CS)])
        return carry
    lax.fori_loop(0, RPT // CS, zdma, 0)

    if with_cnt:
        def zcnt(i, carry):
            cntv[pl.ds(i * 16, 16)] = z16
            return carry
        lax.fori_loop(0, NPAD // 16, zcnt, 0)

    plsc.subcore_barrier()

    gsems = sems[:KB]
    ssems = sems[KB:]

    def body(i, carry):
        c0 = i * KB
        gd = [
            pltpu.async_copy(table.at[src_slab.at[c0 + j]], rows.at[j],
                             gsems[j])
            for j in range(KB)
        ]
        sd = []
        for j in range(KB):
            gd[j].wait()
            sd.append(pltpu.async_copy(rows.at[j],
                                       agg_sh.at[dst_slab.at[c0 + j]],
                                       ssems[j], add=True))
            if with_cnt:
                for k in range(CS // 16):
                    idxk = dst_slab[c0 + j, pl.ds(k * 16, 16)]
                    plsc.addupdate_scatter(cntv, [idxk], ones16)
        for j in range(KB):
            sd[j].wait()
        return carry
    lax.fori_loop(0, NSTEP, body, 0)

    plsc.subcore_barrier()
    pltpu.sync_copy(agg_sh.at[pl.ds(s * RPT, RPT)],
                    agg_out.at[pl.ds(c * NPAD + s * RPT, RPT)])
    if with_cnt:
        pltpu.sync_copy(cntv, cnt_out.at[wid])


def _sc_agg_cnt_body(table, src, dst, agg_out, cnt_out,
                     src_slab, dst_slab, rows, cntv, agg_sh, *sems):
    _sc_agg_impl(True, table, src, dst, agg_out, cnt_out,
                 src_slab, dst_slab, rows, cntv, agg_sh, sems)


def _sc_agg_body(table, src, dst, agg_out,
                 src_slab, dst_slab, rows, agg_sh, *sems):
    _sc_agg_impl(False, table, src, dst, agg_out, None,
                 src_slab, dst_slab, rows, None, agg_sh, sems)


_sc_params = pltpu.CompilerParams(needs_layout_passes=False,
                                  use_tc_tiling_on_sc=False)

_sc_agg_cnt = pl.kernel(
    _sc_agg_cnt_body,
    compiler_params=_sc_params,
    out_type=(jax.ShapeDtypeStruct((NC * NPAD, H), jnp.float32),
              jax.ShapeDtypeStruct((NW, NPAD), jnp.float32)),
    mesh=_mesh,
    scratch_types=[
        pltpu.VMEM((NCHUNK, CS), jnp.int32),
        pltpu.VMEM((NCHUNK, CS), jnp.int32),
        pltpu.VMEM((KB, CS, H), jnp.float32),
        pltpu.VMEM((NPAD,), jnp.float32),
        pltpu.VMEM_SHARED((NPAD, H), jnp.float32),
    ] + [pltpu.SemaphoreType.DMA] * (2 * KB),
)

_sc_agg = pl.kernel(
    _sc_agg_body,
    compiler_params=_sc_params,
    out_type=jax.ShapeDtypeStruct((NC * NPAD, H), jnp.float32),
    mesh=_mesh,
    scratch_types=[
        pltpu.VMEM((NCHUNK, CS), jnp.int32),
        pltpu.VMEM((NCHUNK, CS), jnp.int32),
        pltpu.VMEM((KB, CS, H), jnp.float32),
        pltpu.VMEM_SHARED((NPAD, H), jnp.float32),
    ] + [pltpu.SemaphoreType.DMA] * (2 * KB),
)

P = 2
PD, PH = P * D, P * H
NR = N // P
NPR = NPAD // P
BMP = 1000
GRID = NR // BMP


def _proj_body(x_ref, w_ref, y_ref):
    y_ref[...] = jnp.dot(x_ref[...], w_ref[...],
                         preferred_element_type=jnp.float32)


def _make_proj(din):
    return pl.pallas_call(
        _proj_body,
        grid=(GRID,),
        in_specs=[pl.BlockSpec((BMP, din), lambda i: (i, 0)),
                  pl.BlockSpec((din, PH), lambda i: (0, 0))],
        out_specs=pl.BlockSpec((BMP, PH), lambda i: (i, 0)),
        out_shape=jax.ShapeDtypeStruct((NR, PH), jnp.float32),
    )


_tc_proj_d = _make_proj(PD)
_tc_proj_h = _make_proj(PH)


def _lane_select(inv2):
    lane = lax.broadcasted_iota(jnp.int32, (BMP, PH), 1)
    return jnp.where(lane < H, inv2[:, 0:1], inv2[:, 1:2])


def _mid_body(agg_ref, inv_ref, xr_ref, bl0_ref, g0_ref, be0_ref,
              wl1_ref, y1_ref, h0_ref):
    aggs = agg_ref[0] + agg_ref[1]
    pre = aggs * _lane_select(inv_ref[...]) + bl0_ref[...] + xr_ref[...]
    h0 = jnp.maximum(pre * (g0_ref[...] * INV_BN) + be0_ref[...], 0.0)
    h0_ref[...] = h0
    y1_ref[...] = jnp.dot(h0, wl1_ref[...], preferred_element_type=jnp.float32)


_tc_mid = pl.pallas_call(
    _mid_body,
    grid=(GRID,),
    in_specs=[pl.BlockSpec((NC, BMP, PH), lambda i: (0, i, 0)),
              pl.BlockSpec((BMP, P), lambda i: (i, 0)),
              pl.BlockSpec((BMP, PH), lambda i: (i, 0)),
              pl.BlockSpec((1, PH), lambda i: (0, 0)),
              pl.BlockSpec((1, PH), lambda i: (0, 0)),
              pl.BlockSpec((1, PH), lambda i: (0, 0)),
              pl.BlockSpec((PH, PH), lambda i: (0, 0))],
    out_specs=[pl.BlockSpec((BMP, PH), lambda i: (i, 0)),
               pl.BlockSpec((BMP, PH), lambda i: (i, 0))],
    out_shape=[jax.ShapeDtypeStruct((NR, PH), jnp.float32),
               jax.ShapeDtypeStruct((NR, PH), jnp.float32)],
)


def _head_body(agg_ref, inv_ref, hr_ref, bl1_ref, g1_ref, be1_ref,
               wh1_ref, bh1_ref, wh2_ref, bh2_ref, o_ref):
    aggs = agg_ref[0] + agg_ref[1]
    pre = aggs * _lane_select(inv_ref[...]) + bl1_ref[...] + hr_ref[...]
    h1 = jnp.maximum(pre * (g1_ref[...] * INV_BN) + be1_ref[...], 0.0)
    z = jnp.maximum(
        jnp.dot(h1, wh1_ref[...], preferred_element_type=jnp.float32)
        + bh1_ref[...], 0.0)
    o = jnp.dot(z, wh2_ref[...], preferred_element_type=jnp.float32) + bh2_ref[...]
    o_ref[...] = jax.nn.sigmoid(o)


_tc_head = pl.pallas_call(
    _head_body,
    grid=(GRID,),
    in_specs=[pl.BlockSpec((NC, BMP, PH), lambda i: (0, i, 0)),
              pl.BlockSpec((BMP, P), lambda i: (i, 0)),
              pl.BlockSpec((BMP, PH), lambda i: (i, 0)),
              pl.BlockSpec((1, PH), lambda i: (0, 0)),
              pl.BlockSpec((1, PH), lambda i: (0, 0)),
              pl.BlockSpec((1, PH), lambda i: (0, 0)),
              pl.BlockSpec((PH, P * (H // 2)), lambda i: (0, 0)),
              pl.BlockSpec((1, P * (H // 2)), lambda i: (0, 0)),
              pl.BlockSpec((P * (H // 2), P), lambda i: (0, 0)),
              pl.BlockSpec((1, P), lambda i: (0, 0))],
    out_specs=pl.BlockSpec((BMP, P), lambda i: (i, 0)),
    out_shape=jax.ShapeDtypeStruct((NR, P), jnp.float32),
)


def _blkdiag(w):
    din, dout = w.shape
    z = jnp.zeros((P, din, P, dout), w.dtype)
    z = z.at[0, :, 0, :].set(w).at[1, :, 1, :].set(w)
    return z.reshape(P * din, P * dout)


@jax.jit
def kernel(x, edge_index, Wl0, bl0, Wr0, g0, be0, Wl1, bl1, Wr1, g1, be1,
           Wh1, bh1, Wh2, bh2):
    src = edge_index[0].astype(jnp.int32).reshape(NW * NCHUNK, CS)
    dst = edge_index[1].astype(jnp.int32).reshape(NW * NCHUNK, CS)
    xp = x.reshape(NR, PD)

    y0 = _tc_proj_d(xp, _blkdiag(Wl0))
    agg0, cnt = _sc_agg_cnt(y0.reshape(N, H), src, dst)
    xr0 = _tc_proj_d(xp, _blkdiag(Wr0))
    inv2 = 1.0 / jnp.maximum(jnp.sum(cnt.reshape(NW, NPR, P), axis=0),
                             1.0)
    y1, h0 = _tc_mid(agg0.reshape(NC, NPR, PH), inv2, xr0,
                     jnp.tile(bl0, P).reshape(1, PH),
                     jnp.tile(g0, P).reshape(1, PH),
                     jnp.tile(be0, P).reshape(1, PH),
                     _blkdiag(Wl1))
    agg1 = _sc_agg(y1.reshape(N, H), src, dst)
    hr1 = _tc_proj_h(h0, _blkdiag(Wr1))
    out = _tc_head(agg1.reshape(NC, NPR, PH), inv2, hr1,
                   jnp.tile(bl1, P).reshape(1, PH),
                   jnp.tile(g1, P).reshape(1, PH),
                   jnp.tile(be1, P).reshape(1, PH),
                   _blkdiag(Wh1), jnp.tile(bh1, P).reshape(1, P * (H // 2)),
                   _blkdiag(Wh2), jnp.tile(bh2, P).reshape(1, P))
    return out.reshape(N)


# speedup vs baseline: 17.3679x; 1.0395x over previous
import functools
import math

import jax
import jax.numpy as jnp
from jax import lax
from jax.experimental import pallas as pl
from jax.experimental.pallas import tpu as pltpu
from jax.experimental.pallas import tpu_sc as plsc

N, E, D, H = 10000, 320000, 128, 64
NPAD = 10240
NC, NS = 2, 16
NW = NC * NS
EW = E // NW
CS = 80
NCHUNK = EW // CS
RPT = NPAD // NS
INV_BN = 1.0 / math.sqrt(1.0 + 1e-5)

_mesh = plsc.VectorSubcoreMesh(core_axis_name="c", subcore_axis_name="s")


KB = 5
NSTEP = NCHUNK // KB


def _sc_agg_impl(with_cnt, table, src, dst, agg_out, cnt_out,
                 src_slab, dst_slab, rows, cntv, agg_sh, sems):
    c = lax.axis_index("c")
    s = lax.axis_index("s")
    wid = c * NS + s
    z16 = jnp.zeros((16,), jnp.float32)
    ones16 = jnp.ones((16,), jnp.float32)

    pltpu.sync_copy(src.at[pl.ds(wid * NCHUNK, NCHUNK)], src_slab)
    pltpu.sync_copy(dst.at[pl.ds(wid * NCHUNK, NCHUNK)], dst_slab)

    for r in range(CS):
        for j in range(H // 16):
            rows[0, r, pl.ds(j * 16, 16)] = z16

    def zdma(k, carry):
        pltpu.sync_copy(rows.at[0], agg_sh.at[pl.ds(s * RPT + k * CS, CS)])
        return carry
    lax.fori_loop(0, RPT // CS, zdma, 0)

    if with_cnt:
        def zcnt(i, carry):
            cntv[pl.ds(i * 16, 16)] = z16
            return carry
        lax.fori_loop(0, NPAD // 16, zcnt, 0)

    plsc.subcore_barrier()

    gsems = sems[:KB]
    ssems = sems[KB:]

    def body(i, carry):
        c0 = i * KB
        gd = [
            pltpu.async_copy(table.at[src_slab.at[c0 + j]], rows.at[j],
                             gsems[j])
            for j in range(KB)
        ]
        sd = []
        for j in range(KB):
            gd[j].wait()
            sd.append(pltpu.async_copy(rows.at[j],
                                       agg_sh.at[dst_slab.at[c0 + j]],
                                       ssems[j], add=True))
            if with_cnt:
                for k in range(CS // 16):
                    idxk = dst_slab[c0 + j, pl.ds(k * 16, 16)]
                    plsc.addupdate_scatter(cntv, [idxk], ones16)
        for j in range(KB):
            sd[j].wait()
        return carry
    lax.fori_loop(0, NSTEP, body, 0)

    plsc.subcore_barrier()
    pltpu.sync_copy(agg_sh.at[pl.ds(s * RPT, RPT)],
                    agg_out.at[pl.ds(c * NPAD + s * RPT, RPT)])
    if with_cnt:
        pltpu.sync_copy(cntv, cnt_out.at[wid])


def _sc_agg_cnt_body(table, src, dst, agg_out, cnt_out,
                     src_slab, dst_slab, rows, cntv, agg_sh, *sems):
    _sc_agg_impl(True, table, src, dst, agg_out, cnt_out,
                 src_slab, dst_slab, rows, cntv, agg_sh, sems)


def _sc_agg_body(table, src, dst, agg_out,
                 src_slab, dst_slab, rows, agg_sh, *sems):
    _sc_agg_impl(False, table, src, dst, agg_out, None,
                 src_slab, dst_slab, rows, None, agg_sh, sems)


_sc_params = pltpu.CompilerParams(needs_layout_passes=False,
                                  use_tc_tiling_on_sc=False)

_sc_agg_cnt = pl.kernel(
    _sc_agg_cnt_body,
    compiler_params=_sc_params,
    out_type=(jax.ShapeDtypeStruct((NC * NPAD, H), jnp.float32),
              jax.ShapeDtypeStruct((NW, NPAD), jnp.float32)),
    mesh=_mesh,
    scratch_types=[
        pltpu.VMEM((NCHUNK, CS), jnp.int32),
        pltpu.VMEM((NCHUNK, CS), jnp.int32),
        pltpu.VMEM((KB, CS, H), jnp.float32),
        pltpu.VMEM((NPAD,), jnp.float32),
        pltpu.VMEM_SHARED((NPAD, H), jnp.float32),
    ] + [pltpu.SemaphoreType.DMA] * (2 * KB),
)

_sc_agg = pl.kernel(
    _sc_agg_body,
    compiler_params=_sc_params,
    out_type=jax.ShapeDtypeStruct((NC * NPAD, H), jnp.float32),
    mesh=_mesh,
    scratch_types=[
        pltpu.VMEM((NCHUNK, CS), jnp.int32),
        pltpu.VMEM((NCHUNK, CS), jnp.int32),
        pltpu.VMEM((KB, CS, H), jnp.float32),
        pltpu.VMEM_SHARED((NPAD, H), jnp.float32),
    ] + [pltpu.SemaphoreType.DMA] * (2 * KB),
)

P = 2
PD, PH = P * D, P * H
NR = N // P
NPR = NPAD // P
BMP = 1000
GRID = NR // BMP


def _proj_body(x_ref, w_ref, y_ref):
    y_ref[...] = jnp.dot(x_ref[...], w_ref[...],
                         preferred_element_type=jnp.float32)


def _make_proj(din):
    return pl.pallas_call(
        _proj_body,
        grid=(GRID,),
        in_specs=[pl.BlockSpec((BMP, din), lambda i: (i, 0)),
                  pl.BlockSpec((din, PH), lambda i: (0, 0))],
        out_specs=pl.BlockSpec((BMP, PH), lambda i: (i, 0)),
        out_shape=jax.ShapeDtypeStruct((NR, PH), jnp.float32),
    )


_tc_proj_d = _make_proj(PD)
_tc_proj_h = _make_proj(PH)


def _lane_select(inv2):
    lane = lax.broadcasted_iota(jnp.int32, (BMP, PH), 1)
    return jnp.where(lane < H, inv2[:, 0:1], inv2[:, 1:2])


def _mid_body(agg_ref, inv_ref, xr_ref, bl0_ref, g0_ref, be0_ref,
              wl1_ref, y1_ref, h0_ref):
    aggs = agg_ref[0] + agg_ref[1]
    pre = aggs * _lane_select(inv_ref[...]) + bl0_ref[...] + xr_ref[...]
    h0 = jnp.maximum(pre * (g0_ref[...] * INV_BN) + be0_ref[...], 0.0)
    h0_ref[...] = h0
    y1_ref[...] = jnp.dot(h0, wl1_ref[...], preferred_element_type=jnp.float32)


_tc_mid = pl.pallas_call(
    _mid_body,
    grid=(GRID,),
    in_specs=[pl.BlockSpec((NC, BMP, PH), lambda i: (0, i, 0)),
              pl.BlockSpec((BMP, P), lambda i: (i, 0)),
              pl.BlockSpec((BMP, PH), lambda i: (i, 0)),
              pl.BlockSpec((1, PH), lambda i: (0, 0)),
              pl.BlockSpec((1, PH), lambda i: (0, 0)),
              pl.BlockSpec((1, PH), lambda i: (0, 0)),
              pl.BlockSpec((PH, PH), lambda i: (0, 0))],
    out_specs=[pl.BlockSpec((BMP, PH), lambda i: (i, 0)),
               pl.BlockSpec((BMP, PH), lambda i: (i, 0))],
    out_shape=[jax.ShapeDtypeStruct((NR, PH), jnp.float32),
               jax.ShapeDtypeStruct((NR, PH), jnp.float32)],
)


def _head_body(agg_ref, inv_ref, hr_ref, bl1_ref, g1_ref, be1_ref,
               wh1_ref, bh1_ref, wh2_ref, bh2_ref, o_ref):
    aggs = agg_ref[0] + agg_ref[1]
    pre = aggs * _lane_select(inv_ref[...]) + bl1_ref[...] + hr_ref[...]
    h1 = jnp.maximum(pre * (g1_ref[...] * INV_BN) + be1_ref[...], 0.0)
    z = jnp.maximum(
        jnp.dot(h1, wh1_ref[...], preferred_element_type=jnp.float32)
        + bh1_ref[...], 0.0)
    o = jnp.dot(z, wh2_ref[...], preferred_element_type=jnp.float32) + bh2_ref[...]
    o_ref[...] = jax.nn.sigmoid(o)


_tc_head = pl.pallas_call(
    _head_body,
    grid=(GRID,),
    in_specs=[pl.BlockSpec((NC, BMP, PH), lambda i: (0, i, 0)),
              pl.BlockSpec((BMP, P), lambda i: (i, 0)),
              pl.BlockSpec((BMP, PH), lambda i: (i, 0)),
              pl.BlockSpec((1, PH), lambda i: (0, 0)),
              pl.BlockSpec((1, PH), lambda i: (0, 0)),
              pl.BlockSpec((1, PH), lambda i: (0, 0)),
              pl.BlockSpec((PH, P * (H // 2)), lambda i: (0, 0)),
              pl.BlockSpec((1, P * (H // 2)), lambda i: (0, 0)),
              pl.BlockSpec((P * (H // 2), P), lambda i: (0, 0)),
              pl.BlockSpec((1, P), lambda i: (0, 0))],
    out_specs=pl.BlockSpec((BMP, P), lambda i: (i, 0)),
    out_shape=jax.ShapeDtypeStruct((NR, P), jnp.float32),
)


def _edgeprep_body(ei_ref, s_ref, d_ref):
    s_ref[...] = ei_ref[0, :]
    d_ref[...] = ei_ref[1, :]


_tc_edgeprep = pl.pallas_call(
    _edgeprep_body,
    grid=(1,),
    in_specs=[pl.BlockSpec((2, E), lambda i: (0, 0))],
    out_specs=[pl.BlockSpec((E,), lambda i: (0,)),
               pl.BlockSpec((E,), lambda i: (0,))],
    out_shape=[jax.ShapeDtypeStruct((E,), jnp.int32),
               jax.ShapeDtypeStruct((E,), jnp.int32)],
)


def _blkdiag(w):
    din, dout = w.shape
    z = jnp.zeros((P, din, P, dout), w.dtype)
    z = z.at[0, :, 0, :].set(w).at[1, :, 1, :].set(w)
    return z.reshape(P * din, P * dout)


@jax.jit
def kernel(x, edge_index, Wl0, bl0, Wr0, g0, be0, Wl1, bl1, Wr1, g1, be1,
           Wh1, bh1, Wh2, bh2):
    src1, dst1 = _tc_edgeprep(edge_index.astype(jnp.int32))
    src = src1.reshape(NW * NCHUNK, CS)
    dst = dst1.reshape(NW * NCHUNK, CS)
    xp = x.reshape(NR, PD)

    y0 = _tc_proj_d(xp, _blkdiag(Wl0))
    agg0, cnt = _sc_agg_cnt(y0.reshape(N, H), src, dst)
    xr0 = _tc_proj_d(xp, _blkdiag(Wr0))
    inv2 = 1.0 / jnp.maximum(jnp.sum(cnt.reshape(NW, NPR, P), axis=0),
                             1.0)
    y1, h0 = _tc_mid(agg0.reshape(NC, NPR, PH), inv2, xr0,
                     jnp.tile(bl0, P).reshape(1, PH),
                     jnp.tile(g0, P).reshape(1, PH),
                     jnp.tile(be0, P).reshape(1, PH),
                     _blkdiag(Wl1))
    agg1 = _sc_agg(y1.reshape(N, H), src, dst)
    hr1 = _tc_proj_h(h0, _blkdiag(Wr1))
    out = _tc_head(agg1.reshape(NC, NPR, PH), inv2, hr1,
                   jnp.tile(bl1, P).reshape(1, PH),
                   jnp.tile(g1, P).reshape(1, PH),
                   jnp.tile(be1, P).reshape(1, PH),
                   _blkdiag(Wh1), jnp.tile(bh1, P).reshape(1, P * (H // 2)),
                   _blkdiag(Wh2), jnp.tile(bh2, P).reshape(1, P))
    return out.reshape(N)


# speedup vs baseline: 17.9618x; 1.0342x over previous
import functools
import math

import jax
import jax.numpy as jnp
from jax import lax
from jax.experimental import pallas as pl
from jax.experimental.pallas import tpu as pltpu
from jax.experimental.pallas import tpu_sc as plsc

N, E, D, H = 10000, 320000, 128, 64
NPAD = 10240
NC, NS = 2, 16
NW = NC * NS
EW = E // NW
CS = 80
NCHUNK = EW // CS
RPT = NPAD // NS
INV_BN = 1.0 / math.sqrt(1.0 + 1e-5)

_mesh = plsc.VectorSubcoreMesh(core_axis_name="c", subcore_axis_name="s")


KB = 5
NSTEP = NCHUNK // KB


def _sc_agg_impl(with_cnt, table, src, dst, agg_out, cnt_out,
                 src_slab, dst_slab, rows, cntv, agg_sh, sems):
    c = lax.axis_index("c")
    s = lax.axis_index("s")
    wid = c * NS + s
    z16 = jnp.zeros((16,), jnp.float32)
    ones16 = jnp.ones((16,), jnp.float32)

    pltpu.sync_copy(src.at[pl.ds(wid * NCHUNK, NCHUNK)], src_slab)
    pltpu.sync_copy(dst.at[pl.ds(wid * NCHUNK, NCHUNK)], dst_slab)

    for r in range(CS):
        for j in range(H // 16):
            rows[0, r, pl.ds(j * 16, 16)] = z16

    def zdma(k, carry):
        pltpu.sync_copy(rows.at[0], agg_sh.at[pl.ds(s * RPT + k * CS, CS)])
        return carry
    lax.fori_loop(0, RPT // CS, zdma, 0)

    if with_cnt:
        def zcnt(i, carry):
            cntv[pl.ds(i * 16, 16)] = z16
            return carry
        lax.fori_loop(0, NPAD // 16, zcnt, 0)

    plsc.subcore_barrier()

    gsems = sems[:KB]
    ssems = sems[KB:]

    def body(i, carry):
        c0 = i * KB
        gd = [
            pltpu.async_copy(table.at[src_slab.at[c0 + j]], rows.at[j],
                             gsems[j])
            for j in range(KB)
        ]
        sd = []
        for j in range(KB):
            gd[j].wait()
            sd.append(pltpu.async_copy(rows.at[j],
                                       agg_sh.at[dst_slab.at[c0 + j]],
                                       ssems[j], add=True))
            if with_cnt:
                for k in range(CS // 16):
                    idxk = dst_slab[c0 + j, pl.ds(k * 16, 16)]
                    plsc.addupdate_scatter(cntv, [idxk], ones16)
        for j in range(KB):
            sd[j].wait()
        return carry
    lax.fori_loop(0, NSTEP, body, 0)

    plsc.subcore_barrier()
    pltpu.sync_copy(agg_sh.at[pl.ds(s * RPT, RPT)],
                    agg_out.at[pl.ds(c * NPAD + s * RPT, RPT)])
    if with_cnt:
        pltpu.sync_copy(cntv, cnt_out.at[wid])


def _sc_agg_cnt_body(table, src, dst, agg_out, cnt_out,
                     src_slab, dst_slab, rows, cntv, agg_sh, *sems):
    _sc_agg_impl(True, table, src, dst, agg_out, cnt_out,
                 src_slab, dst_slab, rows, cntv, agg_sh, sems)


def _sc_agg_body(table, src, dst, agg_out,
                 src_slab, dst_slab, rows, agg_sh, *sems):
    _sc_agg_impl(False, table, src, dst, agg_out, None,
                 src_slab, dst_slab, rows, None, agg_sh, sems)


_sc_params = pltpu.CompilerParams(needs_layout_passes=False,
                                  use_tc_tiling_on_sc=False)

_sc_agg_cnt = pl.kernel(
    _sc_agg_cnt_body,
    compiler_params=_sc_params,
    out_type=(jax.ShapeDtypeStruct((NC * NPAD, H), jnp.float32),
              jax.ShapeDtypeStruct((NW, NPAD), jnp.float32)),
    mesh=_mesh,
    scratch_types=[
        pltpu.VMEM((NCHUNK, CS), jnp.int32),
        pltpu.VMEM((NCHUNK, CS), jnp.int32),
        pltpu.VMEM((KB, CS, H), jnp.float32),
        pltpu.VMEM((NPAD,), jnp.float32),
        pltpu.VMEM_SHARED((NPAD, H), jnp.float32),
    ] + [pltpu.SemaphoreType.DMA] * (2 * KB),
)

_sc_agg = pl.kernel(
    _sc_agg_body,
    compiler_params=_sc_params,
    out_type=jax.ShapeDtypeStruct((NC * NPAD, H), jnp.float32),
    mesh=_mesh,
    scratch_types=[
        pltpu.VMEM((NCHUNK, CS), jnp.int32),
        pltpu.VMEM((NCHUNK, CS), jnp.int32),
        pltpu.VMEM((KB, CS, H), jnp.float32),
        pltpu.VMEM_SHARED((NPAD, H), jnp.float32),
    ] + [pltpu.SemaphoreType.DMA] * (2 * KB),
)

P = 2
PD, PH = P * D, P * H
NR = N // P
NPR = NPAD // P
BMP = 1000
GRID = NR // BMP


def _proj_d_body(x_ref, w_ref, y_ref):
    xp = x_ref[...].reshape(BMP, PD)
    y_ref[...] = jnp.dot(xp, w_ref[...], preferred_element_type=jnp.float32)


_tc_proj_d = pl.pallas_call(
    _proj_d_body,
    grid=(GRID,),
    in_specs=[pl.BlockSpec((P * BMP, D), lambda i: (i, 0)),
              pl.BlockSpec((PD, PH), lambda i: (0, 0))],
    out_specs=pl.BlockSpec((BMP, PH), lambda i: (i, 0)),
    out_shape=jax.ShapeDtypeStruct((NR, PH), jnp.float32),
)


def _proj_h_body(x_ref, w_ref, y_ref):
    y_ref[...] = jnp.dot(x_ref[...], w_ref[...],
                         preferred_element_type=jnp.float32)


_tc_proj_h = pl.pallas_call(
    _proj_h_body,
    grid=(GRID,),
    in_specs=[pl.BlockSpec((BMP, PH), lambda i: (i, 0)),
              pl.BlockSpec((PH, PH), lambda i: (0, 0))],
    out_specs=pl.BlockSpec((BMP, PH), lambda i: (i, 0)),
    out_shape=jax.ShapeDtypeStruct((NR, PH), jnp.float32),
)


def _lane_select(inv_ref):
    inv2 = inv_ref[...]
    lane = lax.broadcasted_iota(jnp.int32, (BMP, PH), 1)
    return jnp.where(lane < H, inv2[:, 0:1], inv2[:, 1:2])


def _mid_body(agg_ref, inv_ref, xr_ref, bl0_ref, g0_ref, be0_ref,
              wl1_ref, y1_ref, h0_ref):
    aggs = agg_ref[0] + agg_ref[1]
    pre = aggs * _lane_select(inv_ref) + bl0_ref[...] + xr_ref[...]
    h0 = jnp.maximum(pre * (g0_ref[...] * INV_BN) + be0_ref[...], 0.0)
    h0_ref[...] = h0
    y1_ref[...] = jnp.dot(h0, wl1_ref[...], preferred_element_type=jnp.float32)


_tc_mid = pl.pallas_call(
    _mid_body,
    grid=(GRID,),
    in_specs=[pl.BlockSpec((NC, BMP, PH), lambda i: (0, i, 0)),
              pl.BlockSpec((BMP, P), lambda i: (i, 0)),
              pl.BlockSpec((BMP, PH), lambda i: (i, 0)),
              pl.BlockSpec((1, PH), lambda i: (0, 0)),
              pl.BlockSpec((1, PH), lambda i: (0, 0)),
              pl.BlockSpec((1, PH), lambda i: (0, 0)),
              pl.BlockSpec((PH, PH), lambda i: (0, 0))],
    out_specs=[pl.BlockSpec((BMP, PH), lambda i: (i, 0)),
               pl.BlockSpec((BMP, PH), lambda i: (i, 0))],
    out_shape=[jax.ShapeDtypeStruct((NR, PH), jnp.float32),
               jax.ShapeDtypeStruct((NR, PH), jnp.float32)],
)


def _head_body(agg_ref, inv_ref, hr_ref, bl1_ref, g1_ref, be1_ref,
               wh1_ref, bh1_ref, wh2_ref, bh2_ref, o_ref):
    aggs = agg_ref[0] + agg_ref[1]
    pre = aggs * _lane_select(inv_ref) + bl1_ref[...] + hr_ref[...]
    h1 = jnp.maximum(pre * (g1_ref[...] * INV_BN) + be1_ref[...], 0.0)
    z = jnp.maximum(
        jnp.dot(h1, wh1_ref[...], preferred_element_type=jnp.float32)
        + bh1_ref[...], 0.0)
    o = jnp.dot(z, wh2_ref[...], preferred_element_type=jnp.float32) + bh2_ref[...]
    o_ref[...] = jax.nn.sigmoid(o)


_tc_head = pl.pallas_call(
    _head_body,
    grid=(GRID,),
    in_specs=[pl.BlockSpec((NC, BMP, PH), lambda i: (0, i, 0)),
              pl.BlockSpec((BMP, P), lambda i: (i, 0)),
              pl.BlockSpec((BMP, PH), lambda i: (i, 0)),
              pl.BlockSpec((1, PH), lambda i: (0, 0)),
              pl.BlockSpec((1, PH), lambda i: (0, 0)),
              pl.BlockSpec((1, PH), lambda i: (0, 0)),
              pl.BlockSpec((PH, P * (H // 2)), lambda i: (0, 0)),
              pl.BlockSpec((1, P * (H // 2)), lambda i: (0, 0)),
              pl.BlockSpec((P * (H // 2), P), lambda i: (0, 0)),
              pl.BlockSpec((1, P), lambda i: (0, 0))],
    out_specs=pl.BlockSpec((BMP, P), lambda i: (i, 0)),
    out_shape=jax.ShapeDtypeStruct((NR, P), jnp.float32),
)


def _edgeprep_body(ei_ref, s_ref, d_ref):
    s_ref[...] = ei_ref[0, :]
    d_ref[...] = ei_ref[1, :]


_tc_edgeprep = pl.pallas_call(
    _edgeprep_body,
    grid=(1,),
    in_specs=[pl.BlockSpec((2, E), lambda i: (0, 0))],
    out_specs=[pl.BlockSpec((E,), lambda i: (0,)),
               pl.BlockSpec((E,), lambda i: (0,))],
    out_shape=[jax.ShapeDtypeStruct((E,), jnp.int32),
               jax.ShapeDtypeStruct((E,), jnp.int32)],
)


def _blkdiag(w):
    din, dout = w.shape
    z = jnp.zeros((P, din, P, dout), w.dtype)
    z = z.at[0, :, 0, :].set(w).at[1, :, 1, :].set(w)
    return z.reshape(P * din, P * dout)


@jax.jit
def kernel(x, edge_index, Wl0, bl0, Wr0, g0, be0, Wl1, bl1, Wr1, g1, be1,
           Wh1, bh1, Wh2, bh2):
    src1, dst1 = _tc_edgeprep(edge_index.astype(jnp.int32))
    src = src1.reshape(NW * NCHUNK, CS)
    dst = dst1.reshape(NW * NCHUNK, CS)
    y0 = _tc_proj_d(x, _blkdiag(Wl0))
    agg0, cnt = _sc_agg_cnt(y0.reshape(N, H), src, dst)
    xr0 = _tc_proj_d(x, _blkdiag(Wr0))
    inv = 1.0 / jnp.maximum(jnp.sum(cnt, axis=0), 1.0)
    inv2 = inv.reshape(NPR, P)
    y1, h0 = _tc_mid(agg0.reshape(NC, NPR, PH), inv2, xr0,
                     jnp.tile(bl0, P).reshape(1, PH),
                     jnp.tile(g0, P).reshape(1, PH),
                     jnp.tile(be0, P).reshape(1, PH),
                     _blkdiag(Wl1))
    agg1 = _sc_agg(y1.reshape(N, H), src, dst)
    hr1 = _tc_proj_h(h0, _blkdiag(Wr1))
    out = _tc_head(agg1.reshape(NC, NPR, PH), inv2, hr1,
                   jnp.tile(bl1, P).reshape(1, PH),
                   jnp.tile(g1, P).reshape(1, PH),
                   jnp.tile(be1, P).reshape(1, PH),
                   _blkdiag(Wh1), jnp.tile(bh1, P).reshape(1, P * (H // 2)),
                   _blkdiag(Wh2), jnp.tile(bh2, P).reshape(1, P))
    return out.reshape(N)


# speedup vs baseline: 17.9641x; 1.0001x over previous
import math

import jax
import jax.numpy as jnp
from jax import lax
from jax.experimental import pallas as pl
from jax.experimental.pallas import tpu as pltpu
from jax.experimental.pallas import tpu_sc as plsc

N, E, D, H = 10000, 320000, 128, 64
NPAD = 10240
NC, NS = 2, 16
NW = NC * NS
EW = E // NW
CS = 80
NCHUNK = EW // CS
RPT = NPAD // NS
INV_BN = 1.0 / math.sqrt(1.0 + 1e-5)

_mesh = plsc.VectorSubcoreMesh(core_axis_name="c", subcore_axis_name="s")


KB = 5
NSTEP = NCHUNK // KB


def _sc_agg_impl(with_cnt, table, src, dst, agg_out, cnt_out,
                 src_slab, dst_slab, rows, cntv, agg_sh, sems):
    c = lax.axis_index("c")
    s = lax.axis_index("s")
    wid = c * NS + s
    z16 = jnp.zeros((16,), jnp.float32)
    ones16 = jnp.ones((16,), jnp.float32)

    pltpu.sync_copy(src.at[pl.ds(wid * NCHUNK, NCHUNK)], src_slab)
    pltpu.sync_copy(dst.at[pl.ds(wid * NCHUNK, NCHUNK)], dst_slab)

    for r in range(CS):
        for j in range(H // 16):
            rows[0, r, pl.ds(j * 16, 16)] = z16

    def zdma(k, carry):
        pltpu.sync_copy(rows.at[0], agg_sh.at[pl.ds(s * RPT + k * CS, CS)])
        return carry
    lax.fori_loop(0, RPT // CS, zdma, 0)

    if with_cnt:
        def zcnt(i, carry):
            cntv[pl.ds(i * 16, 16)] = z16
            return carry
        lax.fori_loop(0, NPAD // 16, zcnt, 0)

    plsc.subcore_barrier()

    gsems = sems[:KB]
    ssems = sems[KB:]

    def body(i, carry):
        c0 = i * KB
        gd = [
            pltpu.async_copy(table.at[src_slab.at[c0 + j]], rows.at[j],
                             gsems[j])
            for j in range(KB)
        ]
        sd = []
        for j in range(KB):
            gd[j].wait()
            sd.append(pltpu.async_copy(rows.at[j],
                                       agg_sh.at[dst_slab.at[c0 + j]],
                                       ssems[j], add=True))
            if with_cnt:
                for k in range(CS // 16):
                    idxk = dst_slab[c0 + j, pl.ds(k * 16, 16)]
                    plsc.addupdate_scatter(cntv, [idxk], ones16)
        for j in range(KB):
            sd[j].wait()
        return carry
    lax.fori_loop(0, NSTEP, body, 0)

    plsc.subcore_barrier()
    pltpu.sync_copy(agg_sh.at[pl.ds(s * RPT, RPT)],
                    agg_out.at[pl.ds(c * NPAD + s * RPT, RPT)])
    if with_cnt:
        pltpu.sync_copy(cntv, cnt_out.at[wid])


def _sc_agg_cnt_body(table, src, dst, agg_out, cnt_out,
                     src_slab, dst_slab, rows, cntv, agg_sh, *sems):
    _sc_agg_impl(True, table, src, dst, agg_out, cnt_out,
                 src_slab, dst_slab, rows, cntv, agg_sh, sems)


def _sc_agg_body(table, src, dst, agg_out,
                 src_slab, dst_slab, rows, agg_sh, *sems):
    _sc_agg_impl(False, table, src, dst, agg_out, None,
                 src_slab, dst_slab, rows, None, agg_sh, sems)


_sc_params = pltpu.CompilerParams(needs_layout_passes=False,
                                  use_tc_tiling_on_sc=False)

_sc_agg_cnt = pl.kernel(
    _sc_agg_cnt_body,
    compiler_params=_sc_params,
    out_type=(jax.ShapeDtypeStruct((NC * NPAD, H), jnp.float32),
              jax.ShapeDtypeStruct((NW, NPAD), jnp.float32)),
    mesh=_mesh,
    scratch_types=[
        pltpu.VMEM((NCHUNK, CS), jnp.int32),
        pltpu.VMEM((NCHUNK, CS), jnp.int32),
        pltpu.VMEM((KB, CS, H), jnp.float32),
        pltpu.VMEM((NPAD,), jnp.float32),
        pltpu.VMEM_SHARED((NPAD, H), jnp.float32),
    ] + [pltpu.SemaphoreType.DMA] * (2 * KB),
)

_sc_agg = pl.kernel(
    _sc_agg_body,
    compiler_params=_sc_params,
    out_type=jax.ShapeDtypeStruct((NC * NPAD, H), jnp.float32),
    mesh=_mesh,
    scratch_types=[
        pltpu.VMEM((NCHUNK, CS), jnp.int32),
        pltpu.VMEM((NCHUNK, CS), jnp.int32),
        pltpu.VMEM((KB, CS, H), jnp.float32),
        pltpu.VMEM_SHARED((NPAD, H), jnp.float32),
    ] + [pltpu.SemaphoreType.DMA] * (2 * KB),
)

P = 2
PD, PH = P * D, P * H
NR = N // P
NPR = NPAD // P
BMP = 1000
GRID = NR // BMP


def _proj_d_body(x_ref, w_ref, y_ref):
    xp = x_ref[...].reshape(BMP, PD)
    y_ref[...] = jnp.dot(xp, w_ref[...], preferred_element_type=jnp.float32)


_tc_proj_d = pl.pallas_call(
    _proj_d_body,
    grid=(GRID,),
    in_specs=[pl.BlockSpec((P * BMP, D), lambda i: (i, 0)),
              pl.BlockSpec((PD, PH), lambda i: (0, 0))],
    out_specs=pl.BlockSpec((BMP, PH), lambda i: (i, 0)),
    out_shape=jax.ShapeDtypeStruct((NR, PH), jnp.float32),
)


def _proj_h_body(x_ref, w_ref, y_ref):
    y_ref[...] = jnp.dot(x_ref[...], w_ref[...],
                         preferred_element_type=jnp.float32)


_tc_proj_h = pl.pallas_call(
    _proj_h_body,
    grid=(GRID,),
    in_specs=[pl.BlockSpec((BMP, PH), lambda i: (i, 0)),
              pl.BlockSpec((PH, PH), lambda i: (0, 0))],
    out_specs=pl.BlockSpec((BMP, PH), lambda i: (i, 0)),
    out_shape=jax.ShapeDtypeStruct((NR, PH), jnp.float32),
)


def _lane_select(inv_ref):
    inv2 = inv_ref[...]
    lane = lax.broadcasted_iota(jnp.int32, (BMP, PH), 1)
    return jnp.where(lane < H, inv2[:, 0:1], inv2[:, 1:2])


def _mid_body(agg_ref, inv_ref, xr_ref, bl0_ref, g0_ref, be0_ref,
              wl1_ref, y1_ref, h0_ref):
    aggs = agg_ref[0] + agg_ref[1]
    pre = aggs * _lane_select(inv_ref) + bl0_ref[...] + xr_ref[...]
    h0 = jnp.maximum(pre * (g0_ref[...] * INV_BN) + be0_ref[...], 0.0)
    h0_ref[...] = h0
    y1_ref[...] = jnp.dot(h0, wl1_ref[...], preferred_element_type=jnp.float32)


_tc_mid = pl.pallas_call(
    _mid_body,
    grid=(GRID,),
    in_specs=[pl.BlockSpec((NC, BMP, PH), lambda i: (0, i, 0)),
              pl.BlockSpec((BMP, P), lambda i: (i, 0)),
              pl.BlockSpec((BMP, PH), lambda i: (i, 0)),
              pl.BlockSpec((1, PH), lambda i: (0, 0)),
              pl.BlockSpec((1, PH), lambda i: (0, 0)),
              pl.BlockSpec((1, PH), lambda i: (0, 0)),
              pl.BlockSpec((PH, PH), lambda i: (0, 0))],
    out_specs=[pl.BlockSpec((BMP, PH), lambda i: (i, 0)),
               pl.BlockSpec((BMP, PH), lambda i: (i, 0))],
    out_shape=[jax.ShapeDtypeStruct((NR, PH), jnp.float32),
               jax.ShapeDtypeStruct((NR, PH), jnp.float32)],
)


def _head_body(agg_ref, inv_ref, hr_ref, bl1_ref, g1_ref, be1_ref,
               wh1_ref, bh1_ref, wh2_ref, bh2_ref, o_ref):
    aggs = agg_ref[0] + agg_ref[1]
    pre = aggs * _lane_select(inv_ref) + bl1_ref[...] + hr_ref[...]
    h1 = jnp.maximum(pre * (g1_ref[...] * INV_BN) + be1_ref[...], 0.0)
    z = jnp.maximum(
        jnp.dot(h1, wh1_ref[...], preferred_element_type=jnp.float32)
        + bh1_ref[...], 0.0)
    o = jnp.dot(z, wh2_ref[...], preferred_element_type=jnp.float32) + bh2_ref[...]
    o_ref[...] = jax.nn.sigmoid(o)


_tc_head = pl.pallas_call(
    _head_body,
    grid=(GRID,),
    in_specs=[pl.BlockSpec((NC, BMP, PH), lambda i: (0, i, 0)),
              pl.BlockSpec((BMP, P), lambda i: (i, 0)),
              pl.BlockSpec((BMP, PH), lambda i: (i, 0)),
              pl.BlockSpec((1, PH), lambda i: (0, 0)),
              pl.BlockSpec((1, PH), lambda i: (0, 0)),
              pl.BlockSpec((1, PH), lambda i: (0, 0)),
              pl.BlockSpec((PH, P * (H // 2)), lambda i: (0, 0)),
              pl.BlockSpec((1, P * (H // 2)), lambda i: (0, 0)),
              pl.BlockSpec((P * (H // 2), P), lambda i: (0, 0)),
              pl.BlockSpec((1, P), lambda i: (0, 0))],
    out_specs=pl.BlockSpec((BMP, P), lambda i: (i, 0)),
    out_shape=jax.ShapeDtypeStruct((NR, P), jnp.float32),
)


def _edgeprep_body(ei_ref, s_ref, d_ref):
    s_ref[...] = ei_ref[0, :]
    d_ref[...] = ei_ref[1, :]


_tc_edgeprep = pl.pallas_call(
    _edgeprep_body,
    grid=(1,),
    in_specs=[pl.BlockSpec((2, E), lambda i: (0, 0))],
    out_specs=[pl.BlockSpec((E,), lambda i: (0,)),
               pl.BlockSpec((E,), lambda i: (0,))],
    out_shape=[jax.ShapeDtypeStruct((E,), jnp.int32),
               jax.ShapeDtypeStruct((E,), jnp.int32)],
)


def _blkdiag(w):
    din, dout = w.shape
    z = jnp.zeros((P, din, P, dout), w.dtype)
    z = z.at[0, :, 0, :].set(w).at[1, :, 1, :].set(w)
    return z.reshape(P * din, P * dout)


@jax.jit
def kernel(x, edge_index, Wl0, bl0, Wr0, g0, be0, Wl1, bl1, Wr1, g1, be1,
           Wh1, bh1, Wh2, bh2):
    src1, dst1 = _tc_edgeprep(edge_index.astype(jnp.int32))
    src = src1.reshape(NW * NCHUNK, CS)
    dst = dst1.reshape(NW * NCHUNK, CS)
    y0 = _tc_proj_d(x, _blkdiag(Wl0))
    agg0, cnt = _sc_agg_cnt(y0.reshape(N, H), src, dst)
    xr0 = _tc_proj_d(x, _blkdiag(Wr0))
    inv = 1.0 / jnp.maximum(jnp.sum(cnt, axis=0), 1.0)
    inv2 = inv.reshape(NPR, P)
    y1, h0 = _tc_mid(agg0.reshape(NC, NPR, PH), inv2, xr0,
                     jnp.tile(bl0, P).reshape(1, PH),
                     jnp.tile(g0, P).reshape(1, PH),
                     jnp.tile(be0, P).reshape(1, PH),
                     _blkdiag(Wl1))
    agg1 = _sc_agg(y1.reshape(N, H), src, dst)
    hr1 = _tc_proj_h(h0, _blkdiag(Wr1))
    out = _tc_head(agg1.reshape(NC, NPR, PH), inv2, hr1,
                   jnp.tile(bl1, P).reshape(1, PH),
                   jnp.tile(g1, P).reshape(1, PH),
                   jnp.tile(be1, P).reshape(1, PH),
                   _blkdiag(Wh1), jnp.tile(bh1, P).reshape(1, P * (H // 2)),
                   _blkdiag(Wh2), jnp.tile(bh2, P).reshape(1, P))
    return out.reshape(N)
